# baseline (speedup 1.0000x reference)
import functools

import jax
import jax.numpy as jnp
from jax import lax
from jax.experimental import pallas as pl
from jax.experimental.pallas import tpu as pltpu
from jax.experimental.pallas import tpu_sc as plsc

_BE = 2000


def _node_pre_body(x_ref, w1d_ref, w1s_ref, b1_ref, a_ref, b_ref):
    x = x_ref[...]
    a_ref[...] = jnp.dot(x, w1d_ref[...], preferred_element_type=jnp.float32) + b1_ref[...]
    b_ref[...] = jnp.dot(x, w1s_ref[...], preferred_element_type=jnp.float32)


def _node_pre(x, w1d_t, w1s_t, b1):
    n = x.shape[0]
    hdim = w1d_t.shape[1]
    return pl.pallas_call(
        _node_pre_body,
        out_shape=(jax.ShapeDtypeStruct((n, hdim), jnp.float32),
                   jax.ShapeDtypeStruct((n, hdim), jnp.float32)),
    )(x, w1d_t, w1s_t, b1.reshape(1, -1))


def _node1_body(x_ref, s_ref, c_ref, pw_ref, pb_ref, w1i_ref, sb1_ref, w1j_ref,
                h_ref, a_ref, b_ref):
    x = x_ref[...]
    c = jnp.maximum(c_ref[...], 1.0)
    hin = x * (1.0 + s_ref[...] / c)
    h = jnp.dot(hin, pw_ref[...], preferred_element_type=jnp.float32) + pb_ref[...]
    h_ref[...] = h
    a_ref[...] = jnp.dot(h, w1i_ref[...], preferred_element_type=jnp.float32) + sb1_ref[...]
    b_ref[...] = jnp.dot(h, w1j_ref[...], preferred_element_type=jnp.float32)


def _node1(x, s, cnt, pw_t, pb, w1i_t, sb1, w1j_t):
    n, d = x.shape
    hdim = pw_t.shape[1]
    return pl.pallas_call(
        _node1_body,
        out_shape=(jax.ShapeDtypeStruct((n, hdim), jnp.float32),
                   jax.ShapeDtypeStruct((n, hdim), jnp.float32),
                   jax.ShapeDtypeStruct((n, hdim), jnp.float32)),
    )(x, s, cnt, pw_t, pb.reshape(1, -1), w1i_t, sb1.reshape(1, -1), w1j_t)


def _bn_relu(h, g, b):
    m = jnp.mean(h, axis=0, keepdims=True)
    v = jnp.mean((h - m) ** 2, axis=0, keepdims=True)
    return jnp.maximum((h - m) * jax.lax.rsqrt(v + 1e-5) * g + b, 0.0)


def _node2_body(h_ref, s_ref, c_ref, g_ref, bb_ref, w1i_ref, sb1_ref, w1j_ref,
                h_out_ref, a_ref, b_ref):
    c = jnp.maximum(c_ref[...], 1.0)
    h = h_ref[...] * (1.0 + s_ref[...] / c)
    hn = _bn_relu(h, g_ref[...], bb_ref[...])
    h_out_ref[...] = hn
    a_ref[...] = jnp.dot(hn, w1i_ref[...], preferred_element_type=jnp.float32) + sb1_ref[...]
    b_ref[...] = jnp.dot(hn, w1j_ref[...], preferred_element_type=jnp.float32)


def _node2(h, s, cnt, bn_g, bn_b, w1i_t, sb1, w1j_t):
    n, hdim = h.shape
    return pl.pallas_call(
        _node2_body,
        out_shape=(jax.ShapeDtypeStruct((n, hdim), jnp.float32),
                   jax.ShapeDtypeStruct((n, hdim), jnp.float32),
                   jax.ShapeDtypeStruct((n, hdim), jnp.float32)),
    )(h, s, cnt, bn_g.reshape(1, -1), bn_b.reshape(1, -1),
      w1i_t, sb1.reshape(1, -1), w1j_t)


def _node3_body(h_ref, s_ref, c_ref, g_ref, bb_ref,
                w1_ref, b1_ref, g1_ref, bb1_ref,
                w2_ref, b2_ref, g2_ref, bb2_ref,
                w3_ref, b3_ref, out_ref):
    c = jnp.maximum(c_ref[...], 1.0)
    h = h_ref[...] * (1.0 + s_ref[...] / c)
    hn = _bn_relu(h, g_ref[...], bb_ref[...])
    z = jnp.dot(hn, w1_ref[...], preferred_element_type=jnp.float32) + b1_ref[...]
    z = _bn_relu(z, g1_ref[...], bb1_ref[...])
    z = jnp.dot(z, w2_ref[...], preferred_element_type=jnp.float32) + b2_ref[...]
    z = _bn_relu(z, g2_ref[...], bb2_ref[...])
    out_ref[...] = jnp.dot(z, w3_ref[...], preferred_element_type=jnp.float32) + b3_ref[...]


def _node3(h, s, cnt, bn_g, bn_b, clf):
    n = h.shape[0]
    return pl.pallas_call(
        _node3_body,
        out_shape=jax.ShapeDtypeStruct((n, 1), jnp.float32),
    )(h, s, cnt, bn_g.reshape(1, -1), bn_b.reshape(1, -1),
      clf['W1'].T, clf['b1'].reshape(1, -1), clf['bn1_g'].reshape(1, -1), clf['bn1_b'].reshape(1, -1),
      clf['W2'].T, clf['b2'].reshape(1, -1), clf['bn2_g'].reshape(1, -1), clf['bn2_b'].reshape(1, -1),
      clf['W3'].T, clf['b3'].reshape(1, -1))


def _edge1_body(g_ref, d_ref, freq_ref, ph_ref, wrel_ref, w2_ref, b2_ref, q_ref):
    rel = jnp.cos(d_ref[...] * freq_ref[...] + ph_ref[...])
    hmid = jnp.maximum(
        g_ref[...] + jnp.dot(rel, wrel_ref[...], preferred_element_type=jnp.float32), 0.0)
    p = jnp.tanh(jnp.dot(hmid, w2_ref[...], preferred_element_type=jnp.float32) + b2_ref[...])
    q_ref[...] = 2.0 * p - 1.0


def _edge1(gsum, dts2d, freq, phase, wrel_t, w2_t, b2):
    e, hdim = gsum.shape
    dout = w2_t.shape[1]
    t = freq.shape[0]
    grid = e // _BE
    return pl.pallas_call(
        _edge1_body,
        grid=(grid,),
        in_specs=[
            pl.BlockSpec((_BE, hdim), lambda i: (i, 0)),
            pl.BlockSpec((_BE, 1), lambda i: (i, 0)),
            pl.BlockSpec((1, t), lambda i: (0, 0)),
            pl.BlockSpec((1, t), lambda i: (0, 0)),
            pl.BlockSpec((t, hdim), lambda i: (0, 0)),
            pl.BlockSpec((hdim, dout), lambda i: (0, 0)),
            pl.BlockSpec((1, dout), lambda i: (0, 0)),
        ],
        out_specs=pl.BlockSpec((_BE, dout), lambda i: (i, 0)),
        out_shape=jax.ShapeDtypeStruct((e, dout), jnp.float32),
    )(gsum, dts2d, freq.reshape(1, -1), phase.reshape(1, -1), wrel_t, w2_t,
      b2.reshape(1, -1))


def _edge2_body(g_ref, w2_ref, b2_ref, q_ref):
    hmid = jnp.maximum(g_ref[...], 0.0)
    p = jnp.tanh(jnp.dot(hmid, w2_ref[...], preferred_element_type=jnp.float32) + b2_ref[...])
    q_ref[...] = 2.0 * p - 1.0


def _edge2(gsum, w2_t, b2):
    e, hdim = gsum.shape
    dout = w2_t.shape[1]
    grid = e // _BE
    return pl.pallas_call(
        _edge2_body,
        grid=(grid,),
        in_specs=[
            pl.BlockSpec((_BE, hdim), lambda i: (i, 0)),
            pl.BlockSpec((hdim, dout), lambda i: (0, 0)),
            pl.BlockSpec((1, dout), lambda i: (0, 0)),
        ],
        out_specs=pl.BlockSpec((_BE, dout), lambda i: (i, 0)),
        out_shape=jax.ShapeDtypeStruct((e, dout), jnp.float32),
    )(gsum, w2_t, b2.reshape(1, -1))


def _gather_sum(a, b, dst, src):
    return jnp.take(a, dst, axis=0) + jnp.take(b, src, axis=0)


def _scatter_sum(q, dst, n):
    return jax.ops.segment_sum(q, dst, num_segments=n)


def _edge_counts(dst, n):
    return jax.ops.segment_sum(jnp.ones((dst.shape[0], 1), jnp.float32), dst,
                               num_segments=n)


def kernel(x, dts, params, edge_index):
    src = edge_index[0]
    dst = edge_index[1]
    n, d = x.shape
    e = dst.shape[0]
    t = params['basis_freq'].shape[0]

    w1 = params['tmp_W1']
    w1d_t = w1[:, :d].T
    w1s_t = w1[:, d:2 * d].T
    w1rel_t = w1[:, 2 * d:].T

    cnt = _edge_counts(dst, n)

    a1, b1t = _node_pre(x, w1d_t, w1s_t, params['tmp_b1'])
    g1 = _gather_sum(a1, b1t, dst, src)
    q1 = _edge1(g1, dts.reshape(-1, 1), params['basis_freq'], params['phase'],
                w1rel_t, params['tmp_W2'].T, params['tmp_b2'])
    s1 = _scatter_sum(q1, dst, n)

    smp0, smp1 = params['smp']
    h, a2, b2t = _node1(x, s1, cnt, params['proj_W'].T, params['proj_b'],
                        smp0['W1'][:, :d].T, smp0['b1'], smp0['W1'][:, d:].T)

    g2 = _gather_sum(a2, b2t, dst, src)
    q2 = _edge2(g2, smp0['W2'].T, smp0['b2'])
    s2 = _scatter_sum(q2, dst, n)
    h, a3, b3t = _node2(h, s2, cnt, smp0['bn_g'], smp0['bn_b'],
                        smp1['W1'][:, :d].T, smp1['b1'], smp1['W1'][:, d:].T)

    g3 = _gather_sum(a3, b3t, dst, src)
    q3 = _edge2(g3, smp1['W2'].T, smp1['b2'])
    s3 = _scatter_sum(q3, dst, n)

    return _node3(h, s3, cnt, smp1['bn_g'], smp1['bn_b'], params['clf'])


# speedup vs baseline: 2.6713x; 2.6713x over previous
import functools

import jax
import jax.numpy as jnp
from jax import lax
from jax.experimental import pallas as pl
from jax.experimental.pallas import tpu as pltpu
from jax.experimental.pallas import tpu_sc as plsc

_BE = 2000


def _node_pre_body(x_ref, w1d_ref, w1s_ref, b1_ref, a_ref, b_ref):
    x = x_ref[...]
    a_ref[...] = jnp.dot(x, w1d_ref[...], preferred_element_type=jnp.float32) + b1_ref[...]
    b_ref[...] = jnp.dot(x, w1s_ref[...], preferred_element_type=jnp.float32)


def _node_pre(x, w1d_t, w1s_t, b1):
    n = x.shape[0]
    hdim = w1d_t.shape[1]
    return pl.pallas_call(
        _node_pre_body,
        out_shape=(jax.ShapeDtypeStruct((n, hdim), jnp.float32),
                   jax.ShapeDtypeStruct((n, hdim), jnp.float32)),
    )(x, w1d_t, w1s_t, b1.reshape(1, -1))


def _part_sum(s_ref, c_ref, n):
    sf = s_ref[...]
    cf = c_ref[...]
    s = sf[0:n] + sf[n:]
    c = jnp.maximum(cf[0:n, 0:1] + cf[n:, 0:1], 1.0)
    return s, c


def _node1_body(x_ref, s_ref, c_ref, pw_ref, pb_ref, w1i_ref, sb1_ref, w1j_ref,
                h_ref, a_ref, b_ref):
    x = x_ref[...]
    s, c = _part_sum(s_ref, c_ref, x.shape[0])
    hin = x * (1.0 + s / c)
    h = jnp.dot(hin, pw_ref[...], preferred_element_type=jnp.float32) + pb_ref[...]
    h_ref[...] = h
    a_ref[...] = jnp.dot(h, w1i_ref[...], preferred_element_type=jnp.float32) + sb1_ref[...]
    b_ref[...] = jnp.dot(h, w1j_ref[...], preferred_element_type=jnp.float32)


def _node1(x, s, cnt, pw_t, pb, w1i_t, sb1, w1j_t):
    n, d = x.shape
    hdim = pw_t.shape[1]
    return pl.pallas_call(
        _node1_body,
        out_shape=(jax.ShapeDtypeStruct((n, hdim), jnp.float32),
                   jax.ShapeDtypeStruct((n, hdim), jnp.float32),
                   jax.ShapeDtypeStruct((n, hdim), jnp.float32)),
    )(x, s, cnt, pw_t, pb.reshape(1, -1), w1i_t, sb1.reshape(1, -1), w1j_t)


def _bn_relu(h, g, b):
    m = jnp.mean(h, axis=0, keepdims=True)
    v = jnp.mean((h - m) ** 2, axis=0, keepdims=True)
    return jnp.maximum((h - m) * jax.lax.rsqrt(v + 1e-5) * g + b, 0.0)


def _node2_body(h_ref, s_ref, c_ref, g_ref, bb_ref, w1i_ref, sb1_ref, w1j_ref,
                h_out_ref, a_ref, b_ref):
    s, c = _part_sum(s_ref, c_ref, h_ref.shape[0])
    h = h_ref[...] * (1.0 + s / c)
    hn = _bn_relu(h, g_ref[...], bb_ref[...])
    h_out_ref[...] = hn
    a_ref[...] = jnp.dot(hn, w1i_ref[...], preferred_element_type=jnp.float32) + sb1_ref[...]
    b_ref[...] = jnp.dot(hn, w1j_ref[...], preferred_element_type=jnp.float32)


def _node2(h, s, cnt, bn_g, bn_b, w1i_t, sb1, w1j_t):
    n, hdim = h.shape
    return pl.pallas_call(
        _node2_body,
        out_shape=(jax.ShapeDtypeStruct((n, hdim), jnp.float32),
                   jax.ShapeDtypeStruct((n, hdim), jnp.float32),
                   jax.ShapeDtypeStruct((n, hdim), jnp.float32)),
    )(h, s, cnt, bn_g.reshape(1, -1), bn_b.reshape(1, -1),
      w1i_t, sb1.reshape(1, -1), w1j_t)


def _node3_body(h_ref, s_ref, c_ref, g_ref, bb_ref,
                w1_ref, b1_ref, g1_ref, bb1_ref,
                w2_ref, b2_ref, g2_ref, bb2_ref,
                w3_ref, b3_ref, out_ref):
    s, c = _part_sum(s_ref, c_ref, h_ref.shape[0])
    h = h_ref[...] * (1.0 + s / c)
    hn = _bn_relu(h, g_ref[...], bb_ref[...])
    z = jnp.dot(hn, w1_ref[...], preferred_element_type=jnp.float32) + b1_ref[...]
    z = _bn_relu(z, g1_ref[...], bb1_ref[...])
    z = jnp.dot(z, w2_ref[...], preferred_element_type=jnp.float32) + b2_ref[...]
    z = _bn_relu(z, g2_ref[...], bb2_ref[...])
    out_ref[...] = jnp.dot(z, w3_ref[...], preferred_element_type=jnp.float32) + b3_ref[...]


def _node3(h, s, cnt, bn_g, bn_b, clf):
    n = h.shape[0]
    return pl.pallas_call(
        _node3_body,
        out_shape=jax.ShapeDtypeStruct((n, 1), jnp.float32),
    )(h, s, cnt, bn_g.reshape(1, -1), bn_b.reshape(1, -1),
      clf['W1'].T, clf['b1'].reshape(1, -1), clf['bn1_g'].reshape(1, -1), clf['bn1_b'].reshape(1, -1),
      clf['W2'].T, clf['b2'].reshape(1, -1), clf['bn2_g'].reshape(1, -1), clf['bn2_b'].reshape(1, -1),
      clf['W3'].T, clf['b3'].reshape(1, -1))


def _edge1_body(g_ref, d_ref, freq_ref, ph_ref, wrel_ref, w2_ref, b2_ref, q_ref):
    rel = jnp.cos(d_ref[...] * freq_ref[...] + ph_ref[...])
    hmid = jnp.maximum(
        g_ref[...] + jnp.dot(rel, wrel_ref[...], preferred_element_type=jnp.float32), 0.0)
    p = jnp.tanh(jnp.dot(hmid, w2_ref[...], preferred_element_type=jnp.float32) + b2_ref[...])
    q_ref[...] = 2.0 * p - 1.0


def _edge1(gsum, dts2d, freq, phase, wrel_t, w2_t, b2):
    e, hdim = gsum.shape
    dout = w2_t.shape[1]
    t = freq.shape[0]
    grid = e // _BE
    return pl.pallas_call(
        _edge1_body,
        grid=(grid,),
        in_specs=[
            pl.BlockSpec((_BE, hdim), lambda i: (i, 0)),
            pl.BlockSpec((_BE, 1), lambda i: (i, 0)),
            pl.BlockSpec((1, t), lambda i: (0, 0)),
            pl.BlockSpec((1, t), lambda i: (0, 0)),
            pl.BlockSpec((t, hdim), lambda i: (0, 0)),
            pl.BlockSpec((hdim, dout), lambda i: (0, 0)),
            pl.BlockSpec((1, dout), lambda i: (0, 0)),
        ],
        out_specs=pl.BlockSpec((_BE, dout), lambda i: (i, 0)),
        out_shape=jax.ShapeDtypeStruct((e, dout), jnp.float32),
    )(gsum, dts2d, freq.reshape(1, -1), phase.reshape(1, -1), wrel_t, w2_t,
      b2.reshape(1, -1))


def _edge2_body(g_ref, w2_ref, b2_ref, q_ref):
    hmid = jnp.maximum(g_ref[...], 0.0)
    p = jnp.tanh(jnp.dot(hmid, w2_ref[...], preferred_element_type=jnp.float32) + b2_ref[...])
    q_ref[...] = 2.0 * p - 1.0


def _edge2(gsum, w2_t, b2):
    e, hdim = gsum.shape
    dout = w2_t.shape[1]
    grid = e // _BE
    return pl.pallas_call(
        _edge2_body,
        grid=(grid,),
        in_specs=[
            pl.BlockSpec((_BE, hdim), lambda i: (i, 0)),
            pl.BlockSpec((hdim, dout), lambda i: (0, 0)),
            pl.BlockSpec((1, dout), lambda i: (0, 0)),
        ],
        out_specs=pl.BlockSpec((_BE, dout), lambda i: (i, 0)),
        out_shape=jax.ShapeDtypeStruct((e, dout), jnp.float32),
    )(gsum, w2_t, b2.reshape(1, -1))


_NC = 2
_NS = 16
_NW = _NC * _NS
_IB = 100


def _sc_mesh():
    return plsc.VectorSubcoreMesh(core_axis_name="c", subcore_axis_name="s",
                                  num_cores=_NC, num_subcores=_NS)


_SC_PARAMS = pltpu.CompilerParams(use_tc_tiling_on_sc=False)


_CH = 4


_SB = 624


def _zero_stripe(zbuf, acc, sid, n, dh, zr):
    def zrow(r, _):
        for t in range(dh // 16):
            zbuf[r, pl.ds(t * 16, 16)] = jnp.zeros((16,), jnp.float32)
        return 0

    lax.fori_loop(0, zr, zrow, 0)
    start = sid * _SB

    def zcopy(t, _):
        pltpu.sync_copy(zbuf, acc.at[pl.ds(start + t * zr, zr)])
        return 0

    lax.fori_loop(0, _SB // zr, zcopy, 0)
    tail = n - _NS * _SB

    @pl.when(sid == _NS - 1)
    def _():
        pltpu.sync_copy(zbuf.at[pl.ds(0, tail)], acc.at[pl.ds(_NS * _SB, tail)])


def _copy_out(acc, out_hbm, cid, sid, n):
    start = sid * _SB
    pltpu.sync_copy(acc.at[pl.ds(start, _SB)],
                    out_hbm.at[pl.ds(cid * n + start, _SB)])
    tail = n - _NS * _SB

    @pl.when(sid == _NS - 1)
    def _():
        pltpu.sync_copy(acc.at[pl.ds(_NS * _SB, tail)],
                        out_hbm.at[pl.ds(cid * n + _NS * _SB, tail)])


def _gather_sum(a_tbl, b_tbl, dst3, src3):
    n, dh = a_tbl.shape
    rw = dst3.shape[1]
    ew = rw * _IB
    e = _NW * ew
    _CH = 2 if dh > 128 else 4

    @functools.partial(
        pl.kernel, mesh=_sc_mesh(),
        out_type=jax.ShapeDtypeStruct((e, dh), jnp.float32),
        compiler_params=_SC_PARAMS,
        scratch_types=[
            pltpu.VMEM((rw, _IB), jnp.int32),
            pltpu.VMEM((rw, _IB), jnp.int32),
            pltpu.VMEM((_CH * _IB, dh), jnp.float32),
            pltpu.VMEM((_CH * _IB, dh), jnp.float32),
            pltpu.SemaphoreType.DMA,
            pltpu.SemaphoreType.DMA,
        ],
    )
    def k(a_hbm, b_hbm, dst_hbm, src_hbm, out_hbm, idxa, idxb, bufa, bufb,
          sema, semb):
        wid = lax.axis_index("s") * _NC + lax.axis_index("c")
        pltpu.sync_copy(dst_hbm.at[wid], idxa)
        pltpu.sync_copy(src_hbm.at[wid], idxb)

        def chunk(jj, _):
            cps = []
            for b in range(_CH):
                j = jj * _CH + b
                dsl = pl.ds(b * _IB, _IB)
                cps.append(pltpu.async_copy(a_hbm.at[idxa.at[j]],
                                            bufa.at[dsl], sema))
                cps.append(pltpu.async_copy(b_hbm.at[idxb.at[j]],
                                            bufb.at[dsl], semb))
            for cp in cps:
                cp.wait()

            def row(r, _):
                for t in range(dh // 16):
                    sl = pl.ds(t * 16, 16)
                    bufa[r, sl] = bufa[r, sl] + bufb[r, sl]
                return 0

            lax.fori_loop(0, _CH * _IB, row, 0)
            pltpu.sync_copy(bufa, out_hbm.at[pl.ds(wid * ew + jj * _CH * _IB,
                                                   _CH * _IB)])
            return 0

        lax.fori_loop(0, rw // _CH, chunk, 0)

    return k(a_tbl, b_tbl, dst3, src3)


def _sc_scatter(q, dst3, n):
    e, dh = q.shape
    rw = dst3.shape[1]
    ew = rw * _IB
    zr = 16
    _CH = 4
    cs = 4
    cw = dh // cs

    @functools.partial(
        pl.kernel, mesh=_sc_mesh(),
        out_type=jax.ShapeDtypeStruct((_NC * n, dh), jnp.float32),
        compiler_params=_SC_PARAMS,
        scratch_types=[
            pltpu.VMEM((rw, _IB), jnp.int32),
            pltpu.VMEM((_CH * _IB, cw), jnp.float32),
            pltpu.VMEM((zr, cw), jnp.float32),
            pltpu.VMEM_SHARED((n, cw), jnp.float32),
        ],
    )
    def k(q_hbm, dst_hbm, out_hbm, idx, qbuf, zbuf, acc):
        cid = lax.axis_index("c")
        sid = lax.axis_index("s")
        wid = sid * _NC + cid
        pltpu.sync_copy(dst_hbm.at[wid], idx)
        for p in range(cs):
            csl = pl.ds(p * cw, cw)
            _zero_stripe(zbuf, acc, sid, n, cw, zr)
            plsc.subcore_barrier()

            def chunk(jj, _):
                pltpu.sync_copy(q_hbm.at[pl.ds(wid * ew + jj * _CH * _IB,
                                               _CH * _IB), csl], qbuf)
                for b in range(_CH):
                    pltpu.sync_copy(qbuf.at[pl.ds(b * _IB, _IB)],
                                    acc.at[idx.at[jj * _CH + b]], add=True)
                return 0

            lax.fori_loop(0, rw // _CH, chunk, 0)
            plsc.subcore_barrier()
            start = sid * _SB
            pltpu.sync_copy(acc.at[pl.ds(start, _SB)],
                            out_hbm.at[pl.ds(cid * n + start, _SB), csl])
            tail = n - _NS * _SB

            @pl.when(sid == _NS - 1)
            def _():
                pltpu.sync_copy(acc.at[pl.ds(_NS * _SB, tail)],
                                out_hbm.at[pl.ds(cid * n + _NS * _SB, tail),
                                           csl])
            plsc.subcore_barrier()

    return k(q, dst3)


def _edge_counts(dst3, n):
    rw = dst3.shape[1]
    zr = 16
    dh = 16

    @functools.partial(
        pl.kernel, mesh=_sc_mesh(),
        out_type=jax.ShapeDtypeStruct((_NC * n, dh), jnp.float32),
        compiler_params=_SC_PARAMS,
        scratch_types=[
            pltpu.VMEM((rw, _IB), jnp.int32),
            pltpu.VMEM((_IB, dh), jnp.float32),
            pltpu.VMEM((zr, dh), jnp.float32),
            pltpu.VMEM_SHARED((n, dh), jnp.float32),
        ],
    )
    def k(dst_hbm, out_hbm, idx, ones, zbuf, acc):
        cid = lax.axis_index("c")
        sid = lax.axis_index("s")
        wid = sid * _NC + cid
        pltpu.sync_copy(dst_hbm.at[wid], idx)

        def fill(r, _):
            ones[r, pl.ds(0, 16)] = jnp.ones((16,), jnp.float32)
            return 0

        lax.fori_loop(0, _IB, fill, 0)
        _zero_stripe(zbuf, acc, sid, n, dh, zr)
        plsc.subcore_barrier()

        def chunk(j, _):
            pltpu.sync_copy(ones, acc.at[idx.at[j]], add=True)
            return 0

        lax.fori_loop(0, rw, chunk, 0)
        plsc.subcore_barrier()
        _copy_out(acc, out_hbm, cid, sid, n)

    return k(dst3)


def kernel(x, dts, params, edge_index):
    src = edge_index[0]
    dst = edge_index[1]
    n, d = x.shape
    e = dst.shape[0]
    t = params['basis_freq'].shape[0]

    w1 = params['tmp_W1']
    hid = w1.shape[0]
    hpad = -hid % 128
    w1d_t = jnp.pad(w1[:, :d].T, ((0, 0), (0, hpad)))
    w1s_t = jnp.pad(w1[:, d:2 * d].T, ((0, 0), (0, hpad)))
    w1rel_t = jnp.pad(w1[:, 2 * d:].T, ((0, 0), (0, hpad)))
    tmp_b1 = jnp.pad(params['tmp_b1'], (0, hpad))
    tmp_w2_t = jnp.pad(params['tmp_W2'].T, ((0, hpad), (0, 0)))

    dst2 = dst.reshape(_NW, -1, _IB)
    src2 = src.reshape(_NW, -1, _IB)
    cnt = _edge_counts(dst2, n)

    a1, b1t = _node_pre(x, w1d_t, w1s_t, tmp_b1)
    g1 = _gather_sum(a1, b1t, dst2, src2)
    q1 = _edge1(g1, dts.reshape(-1, 1), params['basis_freq'], params['phase'],
                w1rel_t, tmp_w2_t, params['tmp_b2'])
    s1 = _sc_scatter(q1, dst2, n)

    smp0, smp1 = params['smp']
    h, a2, b2t = _node1(x, s1, cnt, params['proj_W'].T, params['proj_b'],
                        smp0['W1'][:, :d].T, smp0['b1'], smp0['W1'][:, d:].T)

    g2 = _gather_sum(a2, b2t, dst2, src2)
    q2 = _edge2(g2, smp0['W2'].T, smp0['b2'])
    s2 = _sc_scatter(q2, dst2, n)
    h, a3, b3t = _node2(h, s2, cnt, smp0['bn_g'], smp0['bn_b'],
                        smp1['W1'][:, :d].T, smp1['b1'], smp1['W1'][:, d:].T)

    g3 = _gather_sum(a3, b3t, dst2, src2)
    q3 = _edge2(g3, smp1['W2'].T, smp1['b2'])
    s3 = _sc_scatter(q3, dst2, n)

    return _node3(h, s3, cnt, smp1['bn_g'], smp1['bn_b'], params['clf'])


# speedup vs baseline: 2.8454x; 1.0652x over previous
import functools

import jax
import jax.numpy as jnp
from jax import lax
from jax.experimental import pallas as pl
from jax.experimental.pallas import tpu as pltpu
from jax.experimental.pallas import tpu_sc as plsc

_BE = 2000


def _node_pre_body(x_ref, w1d_ref, w1s_ref, b1_ref, a_ref, b_ref):
    x = x_ref[...]
    a_ref[...] = jnp.dot(x, w1d_ref[...], preferred_element_type=jnp.float32) + b1_ref[...]
    b_ref[...] = jnp.dot(x, w1s_ref[...], preferred_element_type=jnp.float32)


def _node_pre(x, w1d_t, w1s_t, b1):
    n = x.shape[0]
    hdim = w1d_t.shape[1]
    return pl.pallas_call(
        _node_pre_body,
        out_shape=(jax.ShapeDtypeStruct((n, hdim), jnp.float32),
                   jax.ShapeDtypeStruct((n, hdim), jnp.float32)),
    )(x, w1d_t, w1s_t, b1.reshape(1, -1))


def _part_sum(s_ref, c_ref, n):
    sf = s_ref[...]
    cf = c_ref[...]
    s = sf[0:n] + sf[n:]
    c = jnp.maximum(cf[0:n, 0:1] + cf[n:, 0:1], 1.0)
    return s, c


def _node1_body(x_ref, s_ref, c_ref, pw_ref, pb_ref, w1i_ref, sb1_ref, w1j_ref,
                h_ref, a_ref, b_ref):
    x = x_ref[...]
    s, c = _part_sum(s_ref, c_ref, x.shape[0])
    hin = x * (1.0 + s / c)
    h = jnp.dot(hin, pw_ref[...], preferred_element_type=jnp.float32) + pb_ref[...]
    h_ref[...] = h
    a_ref[...] = jnp.dot(h, w1i_ref[...], preferred_element_type=jnp.float32) + sb1_ref[...]
    b_ref[...] = jnp.dot(h, w1j_ref[...], preferred_element_type=jnp.float32)


def _node1(x, s, cnt, pw_t, pb, w1i_t, sb1, w1j_t):
    n, d = x.shape
    hdim = pw_t.shape[1]
    return pl.pallas_call(
        _node1_body,
        out_shape=(jax.ShapeDtypeStruct((n, hdim), jnp.float32),
                   jax.ShapeDtypeStruct((n, hdim), jnp.float32),
                   jax.ShapeDtypeStruct((n, hdim), jnp.float32)),
    )(x, s, cnt, pw_t, pb.reshape(1, -1), w1i_t, sb1.reshape(1, -1), w1j_t)


def _bn_relu(h, g, b):
    m = jnp.mean(h, axis=0, keepdims=True)
    v = jnp.mean((h - m) ** 2, axis=0, keepdims=True)
    return jnp.maximum((h - m) * jax.lax.rsqrt(v + 1e-5) * g + b, 0.0)


def _node2_body(h_ref, s_ref, c_ref, g_ref, bb_ref, w1i_ref, sb1_ref, w1j_ref,
                h_out_ref, a_ref, b_ref):
    s, c = _part_sum(s_ref, c_ref, h_ref.shape[0])
    h = h_ref[...] * (1.0 + s / c)
    hn = _bn_relu(h, g_ref[...], bb_ref[...])
    h_out_ref[...] = hn
    a_ref[...] = jnp.dot(hn, w1i_ref[...], preferred_element_type=jnp.float32) + sb1_ref[...]
    b_ref[...] = jnp.dot(hn, w1j_ref[...], preferred_element_type=jnp.float32)


def _node2(h, s, cnt, bn_g, bn_b, w1i_t, sb1, w1j_t):
    n, hdim = h.shape
    return pl.pallas_call(
        _node2_body,
        out_shape=(jax.ShapeDtypeStruct((n, hdim), jnp.float32),
                   jax.ShapeDtypeStruct((n, hdim), jnp.float32),
                   jax.ShapeDtypeStruct((n, hdim), jnp.float32)),
    )(h, s, cnt, bn_g.reshape(1, -1), bn_b.reshape(1, -1),
      w1i_t, sb1.reshape(1, -1), w1j_t)


def _node3_body(h_ref, s_ref, c_ref, g_ref, bb_ref,
                w1_ref, b1_ref, g1_ref, bb1_ref,
                w2_ref, b2_ref, g2_ref, bb2_ref,
                w3_ref, b3_ref, out_ref):
    s, c = _part_sum(s_ref, c_ref, h_ref.shape[0])
    h = h_ref[...] * (1.0 + s / c)
    hn = _bn_relu(h, g_ref[...], bb_ref[...])
    z = jnp.dot(hn, w1_ref[...], preferred_element_type=jnp.float32) + b1_ref[...]
    z = _bn_relu(z, g1_ref[...], bb1_ref[...])
    z = jnp.dot(z, w2_ref[...], preferred_element_type=jnp.float32) + b2_ref[...]
    z = _bn_relu(z, g2_ref[...], bb2_ref[...])
    out_ref[...] = jnp.dot(z, w3_ref[...], preferred_element_type=jnp.float32) + b3_ref[...]


def _node3(h, s, cnt, bn_g, bn_b, clf):
    n = h.shape[0]
    return pl.pallas_call(
        _node3_body,
        out_shape=jax.ShapeDtypeStruct((n, 1), jnp.float32),
    )(h, s, cnt, bn_g.reshape(1, -1), bn_b.reshape(1, -1),
      clf['W1'].T, clf['b1'].reshape(1, -1), clf['bn1_g'].reshape(1, -1), clf['bn1_b'].reshape(1, -1),
      clf['W2'].T, clf['b2'].reshape(1, -1), clf['bn2_g'].reshape(1, -1), clf['bn2_b'].reshape(1, -1),
      clf['W3'].T, clf['b3'].reshape(1, -1))


def _edge1_body(hid, g_ref, d_ref, freq_ref, ph_ref, wrel_ref, w2_ref, b2_ref,
                q_ref):
    g = g_ref[...]
    if hid < g.shape[1]:
        lane = jax.lax.broadcasted_iota(jnp.int32, g.shape, 1)
        g = jnp.where(lane < hid, g, 0.0)
    rel = jnp.cos(d_ref[...] * freq_ref[...] + ph_ref[...])
    hmid = jnp.maximum(
        g + jnp.dot(rel, wrel_ref[...], preferred_element_type=jnp.float32), 0.0)
    p = jnp.tanh(jnp.dot(hmid, w2_ref[...], preferred_element_type=jnp.float32) + b2_ref[...])
    q_ref[...] = 2.0 * p - 1.0


def _edge1(gsum, dts2d, freq, phase, wrel_t, w2_t, b2, hid):
    e, hdim = gsum.shape
    dout = w2_t.shape[1]
    t = freq.shape[0]
    grid = e // _BE
    return pl.pallas_call(
        functools.partial(_edge1_body, hid),
        grid=(grid,),
        in_specs=[
            pl.BlockSpec((_BE, hdim), lambda i: (i, 0)),
            pl.BlockSpec((_BE, 1), lambda i: (i, 0)),
            pl.BlockSpec((1, t), lambda i: (0, 0)),
            pl.BlockSpec((1, t), lambda i: (0, 0)),
            pl.BlockSpec((t, hdim), lambda i: (0, 0)),
            pl.BlockSpec((hdim, dout), lambda i: (0, 0)),
            pl.BlockSpec((1, dout), lambda i: (0, 0)),
        ],
        out_specs=pl.BlockSpec((_BE, dout), lambda i: (i, 0)),
        out_shape=jax.ShapeDtypeStruct((e, dout), jnp.float32),
    )(gsum, dts2d, freq.reshape(1, -1), phase.reshape(1, -1), wrel_t, w2_t,
      b2.reshape(1, -1))


def _edge2_body(g_ref, w2_ref, b2_ref, q_ref):
    hmid = jnp.maximum(g_ref[...], 0.0)
    p = jnp.tanh(jnp.dot(hmid, w2_ref[...], preferred_element_type=jnp.float32) + b2_ref[...])
    q_ref[...] = 2.0 * p - 1.0


def _edge2(gsum, w2_t, b2):
    e, hdim = gsum.shape
    dout = w2_t.shape[1]
    grid = e // _BE
    return pl.pallas_call(
        _edge2_body,
        grid=(grid,),
        in_specs=[
            pl.BlockSpec((_BE, hdim), lambda i: (i, 0)),
            pl.BlockSpec((hdim, dout), lambda i: (0, 0)),
            pl.BlockSpec((1, dout), lambda i: (0, 0)),
        ],
        out_specs=pl.BlockSpec((_BE, dout), lambda i: (i, 0)),
        out_shape=jax.ShapeDtypeStruct((e, dout), jnp.float32),
    )(gsum, w2_t, b2.reshape(1, -1))


_NC = 2
_NS = 16
_NW = _NC * _NS
_IB = 100


def _sc_mesh():
    return plsc.VectorSubcoreMesh(core_axis_name="c", subcore_axis_name="s",
                                  num_cores=_NC, num_subcores=_NS)


_SC_PARAMS = pltpu.CompilerParams(use_tc_tiling_on_sc=False)


_CH = 4


_SB = 624


def _zero_stripe(zbuf, acc, sid, n, dh, zr):
    def zrow(r, _):
        for t in range(dh // 16):
            zbuf[r, pl.ds(t * 16, 16)] = jnp.zeros((16,), jnp.float32)
        return 0

    lax.fori_loop(0, zr, zrow, 0)
    start = sid * _SB

    def zcopy(t, _):
        pltpu.sync_copy(zbuf, acc.at[pl.ds(start + t * zr, zr)])
        return 0

    lax.fori_loop(0, _SB // zr, zcopy, 0)
    tail = n - _NS * _SB

    @pl.when(sid == _NS - 1)
    def _():
        pltpu.sync_copy(zbuf.at[pl.ds(0, tail)], acc.at[pl.ds(_NS * _SB, tail)])


def _copy_out(acc, out_hbm, cid, sid, n):
    start = sid * _SB
    pltpu.sync_copy(acc.at[pl.ds(start, _SB)],
                    out_hbm.at[pl.ds(cid * n + start, _SB)])
    tail = n - _NS * _SB

    @pl.when(sid == _NS - 1)
    def _():
        pltpu.sync_copy(acc.at[pl.ds(_NS * _SB, tail)],
                        out_hbm.at[pl.ds(cid * n + _NS * _SB, tail)])


def _gather_sum(a_tbl, b_tbl, dst3, src3, out_dh=None):
    n, dh = a_tbl.shape
    out_dh = out_dh or dh
    rw = dst3.shape[1]
    ew = rw * _IB
    e = _NW * ew
    _CH = 2 if dh > 128 else 4
    if rw % _CH:
        _CH = 2

    @functools.partial(
        pl.kernel, mesh=_sc_mesh(),
        out_type=jax.ShapeDtypeStruct((e, out_dh), jnp.float32),
        compiler_params=_SC_PARAMS,
        scratch_types=[
            pltpu.VMEM((rw, _IB), jnp.int32),
            pltpu.VMEM((rw, _IB), jnp.int32),
            pltpu.VMEM((_CH * _IB, dh), jnp.float32),
            pltpu.VMEM((_CH * _IB, dh), jnp.float32),
            pltpu.SemaphoreType.DMA,
            pltpu.SemaphoreType.DMA,
        ],
    )
    def k(a_hbm, b_hbm, dst_hbm, src_hbm, out_hbm, idxa, idxb, bufa, bufb,
          sema, semb):
        wid = lax.axis_index("s") * _NC + lax.axis_index("c")
        pltpu.sync_copy(dst_hbm.at[wid], idxa)
        pltpu.sync_copy(src_hbm.at[wid], idxb)

        def chunk(jj, _):
            cps = []
            for b in range(_CH):
                j = jj * _CH + b
                dsl = pl.ds(b * _IB, _IB)
                cps.append(pltpu.async_copy(a_hbm.at[idxa.at[j]],
                                            bufa.at[dsl], sema))
                cps.append(pltpu.async_copy(b_hbm.at[idxb.at[j]],
                                            bufb.at[dsl], semb))
            for cp in cps:
                cp.wait()

            def row(r, _):
                for t in range(dh // 16):
                    sl = pl.ds(t * 16, 16)
                    bufa[r, sl] = bufa[r, sl] + bufb[r, sl]
                return 0

            lax.fori_loop(0, _CH * _IB, row, 0)
            rsl = pl.ds(wid * ew + jj * _CH * _IB, _CH * _IB)
            if out_dh == dh:
                pltpu.sync_copy(bufa, out_hbm.at[rsl])
            else:
                pltpu.sync_copy(bufa, out_hbm.at[rsl, pl.ds(0, dh)])
            return 0

        lax.fori_loop(0, rw // _CH, chunk, 0)

    return k(a_tbl, b_tbl, dst3, src3)


def _sc_scatter(q, dst3, n):
    e, dh = q.shape
    rw = dst3.shape[1]
    ew = rw * _IB
    zr = 16
    _CH = 4
    cs = 4
    cw = dh // cs

    @functools.partial(
        pl.kernel, mesh=_sc_mesh(),
        out_type=jax.ShapeDtypeStruct((_NC * n, dh), jnp.float32),
        compiler_params=_SC_PARAMS,
        scratch_types=[
            pltpu.VMEM((rw, _IB), jnp.int32),
            pltpu.VMEM((_CH * _IB, cw), jnp.float32),
            pltpu.VMEM((zr, cw), jnp.float32),
            pltpu.VMEM_SHARED((n, cw), jnp.float32),
        ],
    )
    def k(q_hbm, dst_hbm, out_hbm, idx, qbuf, zbuf, acc):
        cid = lax.axis_index("c")
        sid = lax.axis_index("s")
        wid = sid * _NC + cid
        pltpu.sync_copy(dst_hbm.at[wid], idx)
        for p in range(cs):
            csl = pl.ds(p * cw, cw)
            _zero_stripe(zbuf, acc, sid, n, cw, zr)
            plsc.subcore_barrier()

            def chunk(jj, _):
                pltpu.sync_copy(q_hbm.at[pl.ds(wid * ew + jj * _CH * _IB,
                                               _CH * _IB), csl], qbuf)
                for b in range(_CH):
                    pltpu.sync_copy(qbuf.at[pl.ds(b * _IB, _IB)],
                                    acc.at[idx.at[jj * _CH + b]], add=True)
                return 0

            lax.fori_loop(0, rw // _CH, chunk, 0)
            plsc.subcore_barrier()
            start = sid * _SB
            pltpu.sync_copy(acc.at[pl.ds(start, _SB)],
                            out_hbm.at[pl.ds(cid * n + start, _SB), csl])
            tail = n - _NS * _SB

            @pl.when(sid == _NS - 1)
            def _():
                pltpu.sync_copy(acc.at[pl.ds(_NS * _SB, tail)],
                                out_hbm.at[pl.ds(cid * n + _NS * _SB, tail),
                                           csl])
            plsc.subcore_barrier()

    return k(q, dst3)


def _edge_counts(dst3, n):
    rw = dst3.shape[1]
    zr = 16
    dh = 16

    @functools.partial(
        pl.kernel, mesh=_sc_mesh(),
        out_type=jax.ShapeDtypeStruct((_NC * n, dh), jnp.float32),
        compiler_params=_SC_PARAMS,
        scratch_types=[
            pltpu.VMEM((rw, _IB), jnp.int32),
            pltpu.VMEM((_IB, dh), jnp.float32),
            pltpu.VMEM((zr, dh), jnp.float32),
            pltpu.VMEM_SHARED((n, dh), jnp.float32),
        ],
    )
    def k(dst_hbm, out_hbm, idx, ones, zbuf, acc):
        cid = lax.axis_index("c")
        sid = lax.axis_index("s")
        wid = sid * _NC + cid
        pltpu.sync_copy(dst_hbm.at[wid], idx)

        def fill(r, _):
            ones[r, pl.ds(0, 16)] = jnp.ones((16,), jnp.float32)
            return 0

        lax.fori_loop(0, _IB, fill, 0)
        _zero_stripe(zbuf, acc, sid, n, dh, zr)
        plsc.subcore_barrier()

        def chunk(j, _):
            pltpu.sync_copy(ones, acc.at[idx.at[j]], add=True)
            return 0

        lax.fori_loop(0, rw, chunk, 0)
        plsc.subcore_barrier()
        _copy_out(acc, out_hbm, cid, sid, n)

    return k(dst3)


def kernel(x, dts, params, edge_index):
    src = edge_index[0]
    dst = edge_index[1]
    n, d = x.shape
    e = dst.shape[0]
    t = params['basis_freq'].shape[0]

    w1 = params['tmp_W1']
    hid = w1.shape[0]
    hpad = -hid % 128
    w1d_t = w1[:, :d].T
    w1s_t = w1[:, d:2 * d].T
    w1rel_t = jnp.pad(w1[:, 2 * d:].T, ((0, 0), (0, hpad)))
    tmp_b1 = params['tmp_b1']
    tmp_w2_t = jnp.pad(params['tmp_W2'].T, ((0, hpad), (0, 0)))

    dst2 = dst.reshape(_NW, -1, _IB)
    src2 = src.reshape(_NW, -1, _IB)
    cnt = _edge_counts(dst2, n)

    a1, b1t = _node_pre(x, w1d_t, w1s_t, tmp_b1)
    g1 = _gather_sum(a1, b1t, dst2, src2, out_dh=hid + hpad)
    q1 = _edge1(g1, dts.reshape(-1, 1), params['basis_freq'], params['phase'],
                w1rel_t, tmp_w2_t, params['tmp_b2'], hid)
    s1 = _sc_scatter(q1, dst2, n)

    smp0, smp1 = params['smp']
    h, a2, b2t = _node1(x, s1, cnt, params['proj_W'].T, params['proj_b'],
                        smp0['W1'][:, :d].T, smp0['b1'], smp0['W1'][:, d:].T)

    g2 = _gather_sum(a2, b2t, dst2, src2)
    q2 = _edge2(g2, smp0['W2'].T, smp0['b2'])
    s2 = _sc_scatter(q2, dst2, n)
    h, a3, b3t = _node2(h, s2, cnt, smp0['bn_g'], smp0['bn_b'],
                        smp1['W1'][:, :d].T, smp1['b1'], smp1['W1'][:, d:].T)

    g3 = _gather_sum(a3, b3t, dst2, src2)
    q3 = _edge2(g3, smp1['W2'].T, smp1['b2'])
    s3 = _sc_scatter(q3, dst2, n)

    return _node3(h, s3, cnt, smp1['bn_g'], smp1['bn_b'], params['clf'])


# speedup vs baseline: 2.9314x; 1.0302x over previous
import functools

import jax
import jax.numpy as jnp
from jax import lax
from jax.experimental import pallas as pl
from jax.experimental.pallas import tpu as pltpu
from jax.experimental.pallas import tpu_sc as plsc

_BE = 2000


def _node_pre_body(x_ref, w1d_ref, w1s_ref, b1_ref, a_ref, b_ref):
    x = x_ref[...]
    a_ref[...] = jnp.dot(x, w1d_ref[...], preferred_element_type=jnp.float32) + b1_ref[...]
    b_ref[...] = jnp.dot(x, w1s_ref[...], preferred_element_type=jnp.float32)


def _node_pre(x, w1d_t, w1s_t, b1):
    n = x.shape[0]
    hdim = w1d_t.shape[1]
    return pl.pallas_call(
        _node_pre_body,
        out_shape=(jax.ShapeDtypeStruct((n, hdim), jnp.float32),
                   jax.ShapeDtypeStruct((n, hdim), jnp.float32)),
    )(x, w1d_t, w1s_t, b1.reshape(1, -1))


def _part_sum(sa_ref, sb_ref, c_ref, n):
    sa = sa_ref[...]
    sb = sb_ref[...]
    cf = c_ref[...]
    s = sa[0:n] + sa[n:] + sb[0:n] + sb[n:]
    c = jnp.maximum(cf[0:n, 0:1] + cf[n:, 0:1], 1.0)
    return s, c


def _node1_body(x_ref, sa_ref, sb_ref, c_ref, pw_ref, pb_ref, w1i_ref, sb1_ref,
                w1j_ref, h_ref, a_ref, b_ref):
    x = x_ref[...]
    s, c = _part_sum(sa_ref, sb_ref, c_ref, x.shape[0])
    hin = x * (1.0 + s / c)
    h = jnp.dot(hin, pw_ref[...], preferred_element_type=jnp.float32) + pb_ref[...]
    h_ref[...] = h
    a_ref[...] = jnp.dot(h, w1i_ref[...], preferred_element_type=jnp.float32) + sb1_ref[...]
    b_ref[...] = jnp.dot(h, w1j_ref[...], preferred_element_type=jnp.float32)


def _node1(x, s, cnt, pw_t, pb, w1i_t, sb1, w1j_t):
    n, d = x.shape
    hdim = pw_t.shape[1]
    return pl.pallas_call(
        _node1_body,
        out_shape=(jax.ShapeDtypeStruct((n, hdim), jnp.float32),
                   jax.ShapeDtypeStruct((n, hdim), jnp.float32),
                   jax.ShapeDtypeStruct((n, hdim), jnp.float32)),
    )(x, s[0], s[1], cnt, pw_t, pb.reshape(1, -1), w1i_t, sb1.reshape(1, -1),
      w1j_t)


def _bn_relu(h, g, b):
    m = jnp.mean(h, axis=0, keepdims=True)
    v = jnp.mean((h - m) ** 2, axis=0, keepdims=True)
    return jnp.maximum((h - m) * jax.lax.rsqrt(v + 1e-5) * g + b, 0.0)


def _node2_body(h_ref, sa_ref, sb_ref, c_ref, g_ref, bb_ref, w1i_ref, sb1_ref,
                w1j_ref, h_out_ref, a_ref, b_ref):
    s, c = _part_sum(sa_ref, sb_ref, c_ref, h_ref.shape[0])
    h = h_ref[...] * (1.0 + s / c)
    hn = _bn_relu(h, g_ref[...], bb_ref[...])
    h_out_ref[...] = hn
    a_ref[...] = jnp.dot(hn, w1i_ref[...], preferred_element_type=jnp.float32) + sb1_ref[...]
    b_ref[...] = jnp.dot(hn, w1j_ref[...], preferred_element_type=jnp.float32)


def _node2(h, s, cnt, bn_g, bn_b, w1i_t, sb1, w1j_t):
    n, hdim = h.shape
    return pl.pallas_call(
        _node2_body,
        out_shape=(jax.ShapeDtypeStruct((n, hdim), jnp.float32),
                   jax.ShapeDtypeStruct((n, hdim), jnp.float32),
                   jax.ShapeDtypeStruct((n, hdim), jnp.float32)),
    )(h, s[0], s[1], cnt, bn_g.reshape(1, -1), bn_b.reshape(1, -1),
      w1i_t, sb1.reshape(1, -1), w1j_t)


def _node3_body(h_ref, sa_ref, sb_ref, c_ref, g_ref, bb_ref,
                w1_ref, b1_ref, g1_ref, bb1_ref,
                w2_ref, b2_ref, g2_ref, bb2_ref,
                w3_ref, b3_ref, out_ref):
    s, c = _part_sum(sa_ref, sb_ref, c_ref, h_ref.shape[0])
    h = h_ref[...] * (1.0 + s / c)
    hn = _bn_relu(h, g_ref[...], bb_ref[...])
    z = jnp.dot(hn, w1_ref[...], preferred_element_type=jnp.float32) + b1_ref[...]
    z = _bn_relu(z, g1_ref[...], bb1_ref[...])
    z = jnp.dot(z, w2_ref[...], preferred_element_type=jnp.float32) + b2_ref[...]
    z = _bn_relu(z, g2_ref[...], bb2_ref[...])
    out_ref[...] = jnp.dot(z, w3_ref[...], preferred_element_type=jnp.float32) + b3_ref[...]


def _node3(h, s, cnt, bn_g, bn_b, clf):
    n = h.shape[0]
    return pl.pallas_call(
        _node3_body,
        out_shape=jax.ShapeDtypeStruct((n, 1), jnp.float32),
    )(h, s[0], s[1], cnt, bn_g.reshape(1, -1), bn_b.reshape(1, -1),
      clf['W1'].T, clf['b1'].reshape(1, -1), clf['bn1_g'].reshape(1, -1), clf['bn1_b'].reshape(1, -1),
      clf['W2'].T, clf['b2'].reshape(1, -1), clf['bn2_g'].reshape(1, -1), clf['bn2_b'].reshape(1, -1),
      clf['W3'].T, clf['b3'].reshape(1, -1))


def _edge1_body(hid, g_ref, d_ref, freq_ref, ph_ref, wrel_ref, w2_ref, b2_ref,
                q_ref):
    g = g_ref[...]
    if hid < g.shape[1]:
        lane = jax.lax.broadcasted_iota(jnp.int32, g.shape, 1)
        g = jnp.where(lane < hid, g, 0.0)
    rel = jnp.cos(d_ref[...] * freq_ref[...] + ph_ref[...])
    hmid = jnp.maximum(
        g + jnp.dot(rel, wrel_ref[...], preferred_element_type=jnp.float32), 0.0)
    p = jnp.tanh(jnp.dot(hmid, w2_ref[...], preferred_element_type=jnp.float32) + b2_ref[...])
    q_ref[...] = 2.0 * p - 1.0


def _edge1(gsum, dts2d, freq, phase, wrel_t, w2_t, b2, hid):
    e, hdim = gsum.shape
    dout = w2_t.shape[1]
    t = freq.shape[0]
    grid = e // _BE
    return pl.pallas_call(
        functools.partial(_edge1_body, hid),
        grid=(grid,),
        in_specs=[
            pl.BlockSpec((_BE, hdim), lambda i: (i, 0)),
            pl.BlockSpec((_BE, 1), lambda i: (i, 0)),
            pl.BlockSpec((1, t), lambda i: (0, 0)),
            pl.BlockSpec((1, t), lambda i: (0, 0)),
            pl.BlockSpec((t, hdim), lambda i: (0, 0)),
            pl.BlockSpec((hdim, dout), lambda i: (0, 0)),
            pl.BlockSpec((1, dout), lambda i: (0, 0)),
        ],
        out_specs=pl.BlockSpec((_BE, dout), lambda i: (i, 0)),
        out_shape=jax.ShapeDtypeStruct((e, dout), jnp.float32),
    )(gsum, dts2d, freq.reshape(1, -1), phase.reshape(1, -1), wrel_t, w2_t,
      b2.reshape(1, -1))


def _edge2_body(g_ref, w2_ref, b2_ref, q_ref):
    hmid = jnp.maximum(g_ref[...], 0.0)
    p = jnp.tanh(jnp.dot(hmid, w2_ref[...], preferred_element_type=jnp.float32) + b2_ref[...])
    q_ref[...] = 2.0 * p - 1.0


def _edge2(gsum, w2_t, b2):
    e, hdim = gsum.shape
    dout = w2_t.shape[1]
    grid = e // _BE
    return pl.pallas_call(
        _edge2_body,
        grid=(grid,),
        in_specs=[
            pl.BlockSpec((_BE, hdim), lambda i: (i, 0)),
            pl.BlockSpec((hdim, dout), lambda i: (0, 0)),
            pl.BlockSpec((1, dout), lambda i: (0, 0)),
        ],
        out_specs=pl.BlockSpec((_BE, dout), lambda i: (i, 0)),
        out_shape=jax.ShapeDtypeStruct((e, dout), jnp.float32),
    )(gsum, w2_t, b2.reshape(1, -1))


_NC = 2
_NS = 16
_NW = _NC * _NS
_IB = 100


def _sc_mesh():
    return plsc.VectorSubcoreMesh(core_axis_name="c", subcore_axis_name="s",
                                  num_cores=_NC, num_subcores=_NS)


_SC_PARAMS = pltpu.CompilerParams(use_tc_tiling_on_sc=False)


_CH = 4


_SB = 624


def _zero_stripe(zbuf, acc, sid, n, dh, zr):
    def zrow(r, _):
        for t in range(dh // 16):
            zbuf[r, pl.ds(t * 16, 16)] = jnp.zeros((16,), jnp.float32)
        return 0

    lax.fori_loop(0, zr, zrow, 0)
    start = sid * _SB

    def zcopy(t, _):
        pltpu.sync_copy(zbuf, acc.at[pl.ds(start + t * zr, zr)])
        return 0

    lax.fori_loop(0, _SB // zr, zcopy, 0)
    tail = n - _NS * _SB

    @pl.when(sid == _NS - 1)
    def _():
        pltpu.sync_copy(zbuf.at[pl.ds(0, tail)], acc.at[pl.ds(_NS * _SB, tail)])


def _copy_out(acc, out_hbm, cid, sid, n):
    start = sid * _SB
    pltpu.sync_copy(acc.at[pl.ds(start, _SB)],
                    out_hbm.at[pl.ds(cid * n + start, _SB)])
    tail = n - _NS * _SB

    @pl.when(sid == _NS - 1)
    def _():
        pltpu.sync_copy(acc.at[pl.ds(_NS * _SB, tail)],
                        out_hbm.at[pl.ds(cid * n + _NS * _SB, tail)])


def _gather_sum(a_tbl, b_tbl, dst3, src3, out_dh=None):
    n, dh = a_tbl.shape
    out_dh = out_dh or dh
    rw = dst3.shape[1]
    ew = rw * _IB
    e = _NW * ew
    _CH = 2 if dh > 128 else 4
    if rw % _CH:
        _CH = 2

    @functools.partial(
        pl.kernel, mesh=_sc_mesh(),
        out_type=jax.ShapeDtypeStruct((e, out_dh), jnp.float32),
        compiler_params=_SC_PARAMS,
        scratch_types=[
            pltpu.VMEM((rw, _IB), jnp.int32),
            pltpu.VMEM((rw, _IB), jnp.int32),
            pltpu.VMEM((_CH * _IB, dh), jnp.float32),
            pltpu.VMEM((_CH * _IB, dh), jnp.float32),
            pltpu.SemaphoreType.DMA,
            pltpu.SemaphoreType.DMA,
        ],
    )
    def k(a_hbm, b_hbm, dst_hbm, src_hbm, out_hbm, idxa, idxb, bufa, bufb,
          sema, semb):
        wid = lax.axis_index("s") * _NC + lax.axis_index("c")
        pltpu.sync_copy(dst_hbm.at[wid], idxa)
        pltpu.sync_copy(src_hbm.at[wid], idxb)

        def chunk(jj, _):
            cps = []
            for b in range(_CH):
                j = jj * _CH + b
                dsl = pl.ds(b * _IB, _IB)
                cps.append(pltpu.async_copy(a_hbm.at[idxa.at[j]],
                                            bufa.at[dsl], sema))
                cps.append(pltpu.async_copy(b_hbm.at[idxb.at[j]],
                                            bufb.at[dsl], semb))
            for cp in cps:
                cp.wait()

            def row(r, _):
                for t in range(dh // 16):
                    sl = pl.ds(t * 16, 16)
                    bufa[r, sl] = bufa[r, sl] + bufb[r, sl]
                return 0

            lax.fori_loop(0, _CH * _IB, row, 0)
            rsl = pl.ds(wid * ew + jj * _CH * _IB, _CH * _IB)
            if out_dh == dh:
                pltpu.sync_copy(bufa, out_hbm.at[rsl])
            else:
                pltpu.sync_copy(bufa, out_hbm.at[rsl, pl.ds(0, dh)])
            return 0

        lax.fori_loop(0, rw // _CH, chunk, 0)

    return k(a_tbl, b_tbl, dst3, src3)


def _sc_scatter(q, dst3, n):
    e, dh = q.shape
    rw = dst3.shape[1]
    ew = rw * _IB
    zr = 16
    _CH = 4 if rw % 4 == 0 else 2
    cs = 4
    cw = dh // cs

    @functools.partial(
        pl.kernel, mesh=_sc_mesh(),
        out_type=jax.ShapeDtypeStruct((_NC * n, dh), jnp.float32),
        compiler_params=_SC_PARAMS,
        scratch_types=[
            pltpu.VMEM((rw, _IB), jnp.int32),
            pltpu.VMEM((_CH * _IB, cw), jnp.float32),
            pltpu.VMEM((zr, cw), jnp.float32),
            pltpu.VMEM_SHARED((n, cw), jnp.float32),
        ],
    )
    def k(q_hbm, dst_hbm, out_hbm, idx, qbuf, zbuf, acc):
        cid = lax.axis_index("c")
        sid = lax.axis_index("s")
        wid = sid * _NC + cid
        pltpu.sync_copy(dst_hbm.at[wid], idx)
        for p in range(cs):
            csl = pl.ds(p * cw, cw)
            _zero_stripe(zbuf, acc, sid, n, cw, zr)
            plsc.subcore_barrier()

            def chunk(jj, _):
                pltpu.sync_copy(q_hbm.at[pl.ds(wid * ew + jj * _CH * _IB,
                                               _CH * _IB), csl], qbuf)
                for b in range(_CH):
                    pltpu.sync_copy(qbuf.at[pl.ds(b * _IB, _IB)],
                                    acc.at[idx.at[jj * _CH + b]], add=True)
                return 0

            lax.fori_loop(0, rw // _CH, chunk, 0)
            plsc.subcore_barrier()
            start = sid * _SB
            pltpu.sync_copy(acc.at[pl.ds(start, _SB)],
                            out_hbm.at[pl.ds(cid * n + start, _SB), csl])
            tail = n - _NS * _SB

            @pl.when(sid == _NS - 1)
            def _():
                pltpu.sync_copy(acc.at[pl.ds(_NS * _SB, tail)],
                                out_hbm.at[pl.ds(cid * n + _NS * _SB, tail),
                                           csl])
            plsc.subcore_barrier()

    return k(q, dst3)


def _edge_counts(dst3, n):
    rw = dst3.shape[1]
    zr = 16
    dh = 16

    @functools.partial(
        pl.kernel, mesh=_sc_mesh(),
        out_type=jax.ShapeDtypeStruct((_NC * n, dh), jnp.float32),
        compiler_params=_SC_PARAMS,
        scratch_types=[
            pltpu.VMEM((rw, _IB), jnp.int32),
            pltpu.VMEM((_IB, dh), jnp.float32),
            pltpu.VMEM((zr, dh), jnp.float32),
            pltpu.VMEM_SHARED((n, dh), jnp.float32),
        ],
    )
    def k(dst_hbm, out_hbm, idx, ones, zbuf, acc):
        cid = lax.axis_index("c")
        sid = lax.axis_index("s")
        wid = sid * _NC + cid
        pltpu.sync_copy(dst_hbm.at[wid], idx)

        def fill(r, _):
            ones[r, pl.ds(0, 16)] = jnp.ones((16,), jnp.float32)
            return 0

        lax.fori_loop(0, _IB, fill, 0)
        _zero_stripe(zbuf, acc, sid, n, dh, zr)
        plsc.subcore_barrier()

        def chunk(j, _):
            pltpu.sync_copy(ones, acc.at[idx.at[j]], add=True)
            return 0

        lax.fori_loop(0, rw, chunk, 0)
        plsc.subcore_barrier()
        _copy_out(acc, out_hbm, cid, sid, n)

    return k(dst3)


def kernel(x, dts, params, edge_index):
    src = edge_index[0]
    dst = edge_index[1]
    n, d = x.shape
    e = dst.shape[0]
    t = params['basis_freq'].shape[0]

    w1 = params['tmp_W1']
    hid = w1.shape[0]
    hpad = -hid % 128
    w1d_t = w1[:, :d].T
    w1s_t = w1[:, d:2 * d].T
    w1rel_t = jnp.pad(w1[:, 2 * d:].T, ((0, 0), (0, hpad)))
    tmp_b1 = params['tmp_b1']
    tmp_w2_t = jnp.pad(params['tmp_W2'].T, ((0, hpad), (0, 0)))

    eh = e // 2
    dts2d = dts.reshape(-1, 1)
    hv = []
    for i in range(2):
        sl = slice(i * eh, (i + 1) * eh)
        hv.append((dst[sl].reshape(_NW, -1, _IB),
                   src[sl].reshape(_NW, -1, _IB), dts2d[sl]))

    dst_full = dst.reshape(_NW, -1, _IB)
    cnt = _edge_counts(dst_full, n)

    a1, b1t = _node_pre(x, w1d_t, w1s_t, tmp_b1)
    s1 = []
    for d3, s3_, dt in hv:
        g = _gather_sum(a1, b1t, d3, s3_, out_dh=hid + hpad)
        q = _edge1(g, dt, params['basis_freq'], params['phase'],
                   w1rel_t, tmp_w2_t, params['tmp_b2'], hid)
        s1.append(_sc_scatter(q, d3, n))

    smp0, smp1 = params['smp']
    h, a2, b2t = _node1(x, s1, cnt, params['proj_W'].T, params['proj_b'],
                        smp0['W1'][:, :d].T, smp0['b1'], smp0['W1'][:, d:].T)

    s2 = []
    for d3, s3_, _ in hv:
        g = _gather_sum(a2, b2t, d3, s3_)
        q = _edge2(g, smp0['W2'].T, smp0['b2'])
        s2.append(_sc_scatter(q, d3, n))
    h, a3, b3t = _node2(h, s2, cnt, smp0['bn_g'], smp0['bn_b'],
                        smp1['W1'][:, :d].T, smp1['b1'], smp1['W1'][:, d:].T)

    s3 = []
    for d3, s3_, _ in hv:
        g = _gather_sum(a3, b3t, d3, s3_)
        q = _edge2(g, smp1['W2'].T, smp1['b2'])
        s3.append(_sc_scatter(q, d3, n))

    return _node3(h, s3, cnt, smp1['bn_g'], smp1['bn_b'], params['clf'])


# speedup vs baseline: 2.9328x; 1.0005x over previous
import functools

import jax
import jax.numpy as jnp
from jax import lax
from jax.experimental import pallas as pl
from jax.experimental.pallas import tpu as pltpu
from jax.experimental.pallas import tpu_sc as plsc

_BE = 2000


def _node_pre_body(x_ref, w1d_ref, w1s_ref, b1_ref, a_ref, b_ref):
    x = x_ref[...]
    a_ref[...] = jnp.dot(x, w1d_ref[...], preferred_element_type=jnp.float32) + b1_ref[...]
    b_ref[...] = jnp.dot(x, w1s_ref[...], preferred_element_type=jnp.float32)


def _node_pre(x, w1d_t, w1s_t, b1):
    n = x.shape[0]
    hdim = w1d_t.shape[1]
    return pl.pallas_call(
        _node_pre_body,
        out_shape=(jax.ShapeDtypeStruct((n, hdim), jnp.float32),
                   jax.ShapeDtypeStruct((n, hdim), jnp.float32)),
    )(x, w1d_t, w1s_t, b1.reshape(1, -1))


def _part_sum(sa_ref, sb_ref, c_ref, n):
    sa = sa_ref[...]
    sb = sb_ref[...]
    cf = c_ref[...]
    s = sa[0:n] + sa[n:] + sb[0:n] + sb[n:]
    c = jnp.maximum(cf[0:n, 0:1] + cf[n:, 0:1], 1.0)
    return s, c


def _node1_body(x_ref, sa_ref, sb_ref, c_ref, pw_ref, pb_ref, w1i_ref, sb1_ref,
                w1j_ref, h_ref, a_ref, b_ref):
    x = x_ref[...]
    s, c = _part_sum(sa_ref, sb_ref, c_ref, x.shape[0])
    hin = x * (1.0 + s / c)
    h = jnp.dot(hin, pw_ref[...], preferred_element_type=jnp.float32) + pb_ref[...]
    h_ref[...] = h
    a_ref[...] = jnp.dot(h, w1i_ref[...], preferred_element_type=jnp.float32) + sb1_ref[...]
    b_ref[...] = jnp.dot(h, w1j_ref[...], preferred_element_type=jnp.float32)


def _node1(x, s, cnt, pw_t, pb, w1i_t, sb1, w1j_t):
    n, d = x.shape
    hdim = pw_t.shape[1]
    return pl.pallas_call(
        _node1_body,
        out_shape=(jax.ShapeDtypeStruct((n, hdim), jnp.float32),
                   jax.ShapeDtypeStruct((n, hdim), jnp.float32),
                   jax.ShapeDtypeStruct((n, hdim), jnp.float32)),
    )(x, s[0], s[1], cnt, pw_t, pb.reshape(1, -1), w1i_t, sb1.reshape(1, -1),
      w1j_t)


def _bn_relu(h, g, b):
    m = jnp.mean(h, axis=0, keepdims=True)
    v = jnp.mean((h - m) ** 2, axis=0, keepdims=True)
    return jnp.maximum((h - m) * jax.lax.rsqrt(v + 1e-5) * g + b, 0.0)


def _node2_body(h_ref, sa_ref, sb_ref, c_ref, g_ref, bb_ref, w1i_ref, sb1_ref,
                w1j_ref, h_out_ref, a_ref, b_ref):
    s, c = _part_sum(sa_ref, sb_ref, c_ref, h_ref.shape[0])
    h = h_ref[...] * (1.0 + s / c)
    hn = _bn_relu(h, g_ref[...], bb_ref[...])
    h_out_ref[...] = hn
    a_ref[...] = jnp.dot(hn, w1i_ref[...], preferred_element_type=jnp.float32) + sb1_ref[...]
    b_ref[...] = jnp.dot(hn, w1j_ref[...], preferred_element_type=jnp.float32)


def _node2(h, s, cnt, bn_g, bn_b, w1i_t, sb1, w1j_t):
    n, hdim = h.shape
    return pl.pallas_call(
        _node2_body,
        out_shape=(jax.ShapeDtypeStruct((n, hdim), jnp.float32),
                   jax.ShapeDtypeStruct((n, hdim), jnp.float32),
                   jax.ShapeDtypeStruct((n, hdim), jnp.float32)),
    )(h, s[0], s[1], cnt, bn_g.reshape(1, -1), bn_b.reshape(1, -1),
      w1i_t, sb1.reshape(1, -1), w1j_t)


def _node3_body(h_ref, sa_ref, sb_ref, c_ref, g_ref, bb_ref,
                w1_ref, b1_ref, g1_ref, bb1_ref,
                w2_ref, b2_ref, g2_ref, bb2_ref,
                w3_ref, b3_ref, out_ref):
    s, c = _part_sum(sa_ref, sb_ref, c_ref, h_ref.shape[0])
    h = h_ref[...] * (1.0 + s / c)
    hn = _bn_relu(h, g_ref[...], bb_ref[...])
    z = jnp.dot(hn, w1_ref[...], preferred_element_type=jnp.float32) + b1_ref[...]
    z = _bn_relu(z, g1_ref[...], bb1_ref[...])
    z = jnp.dot(z, w2_ref[...], preferred_element_type=jnp.float32) + b2_ref[...]
    z = _bn_relu(z, g2_ref[...], bb2_ref[...])
    out_ref[...] = jnp.dot(z, w3_ref[...], preferred_element_type=jnp.float32) + b3_ref[...]


def _node3(h, s, cnt, bn_g, bn_b, clf):
    n = h.shape[0]
    return pl.pallas_call(
        _node3_body,
        out_shape=jax.ShapeDtypeStruct((n, 1), jnp.float32),
    )(h, s[0], s[1], cnt, bn_g.reshape(1, -1), bn_b.reshape(1, -1),
      clf['W1'].T, clf['b1'].reshape(1, -1), clf['bn1_g'].reshape(1, -1), clf['bn1_b'].reshape(1, -1),
      clf['W2'].T, clf['b2'].reshape(1, -1), clf['bn2_g'].reshape(1, -1), clf['bn2_b'].reshape(1, -1),
      clf['W3'].T, clf['b3'].reshape(1, -1))


def _edge1_body(hid, g_ref, d_ref, freq_ref, ph_ref, wrel_ref, w2_ref, b2_ref,
                q_ref):
    g = g_ref[...]
    if hid < g.shape[1]:
        lane = jax.lax.broadcasted_iota(jnp.int32, g.shape, 1)
        g = jnp.where(lane < hid, g, 0.0)
    rel = jnp.cos(d_ref[...] * freq_ref[...] + ph_ref[...])
    hmid = jnp.maximum(
        g + jnp.dot(rel, wrel_ref[...], preferred_element_type=jnp.float32), 0.0)
    p = jnp.tanh(jnp.dot(hmid.astype(jnp.bfloat16),
                         w2_ref[...].astype(jnp.bfloat16),
                         preferred_element_type=jnp.float32) + b2_ref[...])
    q_ref[...] = 2.0 * p - 1.0


def _edge1(gsum, dts2d, freq, phase, wrel_t, w2_t, b2, hid):
    e, hdim = gsum.shape
    dout = w2_t.shape[1]
    t = freq.shape[0]
    grid = e // _BE
    return pl.pallas_call(
        functools.partial(_edge1_body, hid),
        grid=(grid,),
        in_specs=[
            pl.BlockSpec((_BE, hdim), lambda i: (i, 0)),
            pl.BlockSpec((_BE, 1), lambda i: (i, 0)),
            pl.BlockSpec((1, t), lambda i: (0, 0)),
            pl.BlockSpec((1, t), lambda i: (0, 0)),
            pl.BlockSpec((t, hdim), lambda i: (0, 0)),
            pl.BlockSpec((hdim, dout), lambda i: (0, 0)),
            pl.BlockSpec((1, dout), lambda i: (0, 0)),
        ],
        out_specs=pl.BlockSpec((_BE, dout), lambda i: (i, 0)),
        out_shape=jax.ShapeDtypeStruct((e, dout), jnp.float32),
    )(gsum, dts2d, freq.reshape(1, -1), phase.reshape(1, -1), wrel_t, w2_t,
      b2.reshape(1, -1))


def _edge2_body(g_ref, w2_ref, b2_ref, q_ref):
    hmid = jnp.maximum(g_ref[...], 0.0)
    p = jnp.tanh(jnp.dot(hmid.astype(jnp.bfloat16),
                         w2_ref[...].astype(jnp.bfloat16),
                         preferred_element_type=jnp.float32) + b2_ref[...])
    q_ref[...] = 2.0 * p - 1.0


def _edge2(gsum, w2_t, b2):
    e, hdim = gsum.shape
    dout = w2_t.shape[1]
    grid = e // _BE
    return pl.pallas_call(
        _edge2_body,
        grid=(grid,),
        in_specs=[
            pl.BlockSpec((_BE, hdim), lambda i: (i, 0)),
            pl.BlockSpec((hdim, dout), lambda i: (0, 0)),
            pl.BlockSpec((1, dout), lambda i: (0, 0)),
        ],
        out_specs=pl.BlockSpec((_BE, dout), lambda i: (i, 0)),
        out_shape=jax.ShapeDtypeStruct((e, dout), jnp.float32),
    )(gsum, w2_t, b2.reshape(1, -1))


_NC = 2
_NS = 16
_NW = _NC * _NS
_IB = 100


def _sc_mesh():
    return plsc.VectorSubcoreMesh(core_axis_name="c", subcore_axis_name="s",
                                  num_cores=_NC, num_subcores=_NS)


_SC_PARAMS = pltpu.CompilerParams(use_tc_tiling_on_sc=False)


_CH = 4


_SB = 624


def _zero_stripe(zbuf, acc, sid, n, dh, zr):
    def zrow(r, _):
        for t in range(dh // 16):
            zbuf[r, pl.ds(t * 16, 16)] = jnp.zeros((16,), jnp.float32)
        return 0

    lax.fori_loop(0, zr, zrow, 0)
    start = sid * _SB

    def zcopy(t, _):
        pltpu.sync_copy(zbuf, acc.at[pl.ds(start + t * zr, zr)])
        return 0

    lax.fori_loop(0, _SB // zr, zcopy, 0)
    tail = n - _NS * _SB

    @pl.when(sid == _NS - 1)
    def _():
        pltpu.sync_copy(zbuf.at[pl.ds(0, tail)], acc.at[pl.ds(_NS * _SB, tail)])


def _copy_out(acc, out_hbm, cid, sid, n):
    start = sid * _SB
    pltpu.sync_copy(acc.at[pl.ds(start, _SB)],
                    out_hbm.at[pl.ds(cid * n + start, _SB)])
    tail = n - _NS * _SB

    @pl.when(sid == _NS - 1)
    def _():
        pltpu.sync_copy(acc.at[pl.ds(_NS * _SB, tail)],
                        out_hbm.at[pl.ds(cid * n + _NS * _SB, tail)])


def _gather_sum(a_tbl, b_tbl, dst3, src3, out_dh=None):
    n, dh = a_tbl.shape
    out_dh = out_dh or dh
    rw = dst3.shape[1]
    ew = rw * _IB
    e = _NW * ew
    _CH = 2 if dh > 128 else 4
    if rw % _CH:
        _CH = 2

    @functools.partial(
        pl.kernel, mesh=_sc_mesh(),
        out_type=jax.ShapeDtypeStruct((e, out_dh), jnp.float32),
        compiler_params=_SC_PARAMS,
        scratch_types=[
            pltpu.VMEM((rw, _IB), jnp.int32),
            pltpu.VMEM((rw, _IB), jnp.int32),
            pltpu.VMEM((_CH * _IB, dh), jnp.float32),
            pltpu.VMEM((_CH * _IB, dh), jnp.float32),
            pltpu.SemaphoreType.DMA,
            pltpu.SemaphoreType.DMA,
        ],
    )
    def k(a_hbm, b_hbm, dst_hbm, src_hbm, out_hbm, idxa, idxb, bufa, bufb,
          sema, semb):
        wid = lax.axis_index("s") * _NC + lax.axis_index("c")
        pltpu.sync_copy(dst_hbm.at[wid], idxa)
        pltpu.sync_copy(src_hbm.at[wid], idxb)

        def chunk(jj, _):
            cps = []
            for b in range(_CH):
                j = jj * _CH + b
                dsl = pl.ds(b * _IB, _IB)
                cps.append(pltpu.async_copy(a_hbm.at[idxa.at[j]],
                                            bufa.at[dsl], sema))
                cps.append(pltpu.async_copy(b_hbm.at[idxb.at[j]],
                                            bufb.at[dsl], semb))
            for cp in cps:
                cp.wait()

            def row(r, _):
                for t in range(dh // 16):
                    sl = pl.ds(t * 16, 16)
                    bufa[r, sl] = bufa[r, sl] + bufb[r, sl]
                return 0

            lax.fori_loop(0, _CH * _IB, row, 0)
            rsl = pl.ds(wid * ew + jj * _CH * _IB, _CH * _IB)
            if out_dh == dh:
                pltpu.sync_copy(bufa, out_hbm.at[rsl])
            else:
                pltpu.sync_copy(bufa, out_hbm.at[rsl, pl.ds(0, dh)])
            return 0

        lax.fori_loop(0, rw // _CH, chunk, 0)

    return k(a_tbl, b_tbl, dst3, src3)


def _sc_scatter(q, dst3, n):
    e, dh = q.shape
    rw = dst3.shape[1]
    ew = rw * _IB
    zr = 16
    _CH = 4 if rw % 4 == 0 else 2
    cs = 4
    cw = dh // cs

    @functools.partial(
        pl.kernel, mesh=_sc_mesh(),
        out_type=jax.ShapeDtypeStruct((_NC * n, dh), jnp.float32),
        compiler_params=_SC_PARAMS,
        scratch_types=[
            pltpu.VMEM((rw, _IB), jnp.int32),
            pltpu.VMEM((_CH * _IB, cw), jnp.float32),
            pltpu.VMEM((zr, cw), jnp.float32),
            pltpu.VMEM_SHARED((n, cw), jnp.float32),
        ],
    )
    def k(q_hbm, dst_hbm, out_hbm, idx, qbuf, zbuf, acc):
        cid = lax.axis_index("c")
        sid = lax.axis_index("s")
        wid = sid * _NC + cid
        pltpu.sync_copy(dst_hbm.at[wid], idx)
        for p in range(cs):
            csl = pl.ds(p * cw, cw)
            _zero_stripe(zbuf, acc, sid, n, cw, zr)
            plsc.subcore_barrier()

            def chunk(jj, _):
                pltpu.sync_copy(q_hbm.at[pl.ds(wid * ew + jj * _CH * _IB,
                                               _CH * _IB), csl], qbuf)
                for b in range(_CH):
                    pltpu.sync_copy(qbuf.at[pl.ds(b * _IB, _IB)],
                                    acc.at[idx.at[jj * _CH + b]], add=True)
                return 0

            lax.fori_loop(0, rw // _CH, chunk, 0)
            plsc.subcore_barrier()
            start = sid * _SB
            pltpu.sync_copy(acc.at[pl.ds(start, _SB)],
                            out_hbm.at[pl.ds(cid * n + start, _SB), csl])
            tail = n - _NS * _SB

            @pl.when(sid == _NS - 1)
            def _():
                pltpu.sync_copy(acc.at[pl.ds(_NS * _SB, tail)],
                                out_hbm.at[pl.ds(cid * n + _NS * _SB, tail),
                                           csl])
            plsc.subcore_barrier()

    return k(q, dst3)


def _edge_counts(dst3, n):
    rw = dst3.shape[1]
    zr = 16
    dh = 16

    @functools.partial(
        pl.kernel, mesh=_sc_mesh(),
        out_type=jax.ShapeDtypeStruct((_NC * n, dh), jnp.float32),
        compiler_params=_SC_PARAMS,
        scratch_types=[
            pltpu.VMEM((rw, _IB), jnp.int32),
            pltpu.VMEM((_IB, dh), jnp.float32),
            pltpu.VMEM((zr, dh), jnp.float32),
            pltpu.VMEM_SHARED((n, dh), jnp.float32),
        ],
    )
    def k(dst_hbm, out_hbm, idx, ones, zbuf, acc):
        cid = lax.axis_index("c")
        sid = lax.axis_index("s")
        wid = sid * _NC + cid
        pltpu.sync_copy(dst_hbm.at[wid], idx)

        def fill(r, _):
            ones[r, pl.ds(0, 16)] = jnp.ones((16,), jnp.float32)
            return 0

        lax.fori_loop(0, _IB, fill, 0)
        _zero_stripe(zbuf, acc, sid, n, dh, zr)
        plsc.subcore_barrier()

        def chunk(j, _):
            pltpu.sync_copy(ones, acc.at[idx.at[j]], add=True)
            return 0

        lax.fori_loop(0, rw, chunk, 0)
        plsc.subcore_barrier()
        _copy_out(acc, out_hbm, cid, sid, n)

    return k(dst3)


def kernel(x, dts, params, edge_index):
    src = edge_index[0]
    dst = edge_index[1]
    n, d = x.shape
    e = dst.shape[0]
    t = params['basis_freq'].shape[0]

    w1 = params['tmp_W1']
    hid = w1.shape[0]
    hpad = -hid % 128
    w1d_t = w1[:, :d].T
    w1s_t = w1[:, d:2 * d].T
    w1rel_t = jnp.pad(w1[:, 2 * d:].T, ((0, 0), (0, hpad)))
    tmp_b1 = params['tmp_b1']
    tmp_w2_t = jnp.pad(params['tmp_W2'].T, ((0, hpad), (0, 0)))

    eh = e // 2
    dts2d = dts.reshape(-1, 1)
    hv = []
    for i in range(2):
        sl = slice(i * eh, (i + 1) * eh)
        hv.append((dst[sl].reshape(_NW, -1, _IB),
                   src[sl].reshape(_NW, -1, _IB), dts2d[sl]))

    dst_full = dst.reshape(_NW, -1, _IB)
    cnt = _edge_counts(dst_full, n)

    a1, b1t = _node_pre(x, w1d_t, w1s_t, tmp_b1)
    s1 = []
    for d3, s3_, dt in hv:
        g = _gather_sum(a1, b1t, d3, s3_, out_dh=hid + hpad)
        q = _edge1(g, dt, params['basis_freq'], params['phase'],
                   w1rel_t, tmp_w2_t, params['tmp_b2'], hid)
        s1.append(_sc_scatter(q, d3, n))

    smp0, smp1 = params['smp']
    h, a2, b2t = _node1(x, s1, cnt, params['proj_W'].T, params['proj_b'],
                        smp0['W1'][:, :d].T, smp0['b1'], smp0['W1'][:, d:].T)

    s2 = []
    for d3, s3_, _ in hv:
        g = _gather_sum(a2, b2t, d3, s3_)
        q = _edge2(g, smp0['W2'].T, smp0['b2'])
        s2.append(_sc_scatter(q, d3, n))
    h, a3, b3t = _node2(h, s2, cnt, smp0['bn_g'], smp0['bn_b'],
                        smp1['W1'][:, :d].T, smp1['b1'], smp1['W1'][:, d:].T)

    s3 = []
    for d3, s3_, _ in hv:
        g = _gather_sum(a3, b3t, d3, s3_)
        q = _edge2(g, smp1['W2'].T, smp1['b2'])
        s3.append(_sc_scatter(q, d3, n))

    return _node3(h, s3, cnt, smp1['bn_g'], smp1['bn_b'], params['clf'])


# speedup vs baseline: 3.2805x; 1.1185x over previous
import functools

import jax
import jax.numpy as jnp
from jax import lax
from jax.experimental import pallas as pl
from jax.experimental.pallas import tpu as pltpu
from jax.experimental.pallas import tpu_sc as plsc

_BE = 2000


def _node_pre_body(x_ref, w1d_ref, w1s_ref, b1_ref, a_ref, b_ref):
    x = x_ref[...]
    a_ref[...] = jnp.dot(x, w1d_ref[...], preferred_element_type=jnp.float32) + b1_ref[...]
    b_ref[...] = jnp.dot(x, w1s_ref[...], preferred_element_type=jnp.float32)


def _node_pre(x, w1d_t, w1s_t, b1):
    n = x.shape[0]
    hdim = w1d_t.shape[1]
    return pl.pallas_call(
        _node_pre_body,
        out_shape=(jax.ShapeDtypeStruct((n, hdim), jnp.float32),
                   jax.ShapeDtypeStruct((n, hdim), jnp.float32)),
    )(x, w1d_t, w1s_t, b1.reshape(1, -1))


def _part_sum(sa_ref, sb_ref, c_ref, n):
    sa = sa_ref[...]
    sb = sb_ref[...]
    cf = c_ref[...]
    s = sa[0:n] + sa[n:] + sb[0:n] + sb[n:]
    c = jnp.maximum(cf[0:n, 0:1] + cf[n:, 0:1], 1.0)
    return s, c


def _node1_body(x_ref, sa_ref, sb_ref, c_ref, pw_ref, pb_ref, w1i_ref, sb1_ref,
                w1j_ref, h_ref, a_ref, b_ref):
    x = x_ref[...]
    s, c = _part_sum(sa_ref, sb_ref, c_ref, x.shape[0])
    hin = x * (1.0 + s / c)
    h = jnp.dot(hin, pw_ref[...], preferred_element_type=jnp.float32) + pb_ref[...]
    h_ref[...] = h
    a_ref[...] = jnp.dot(h, w1i_ref[...], preferred_element_type=jnp.float32) + sb1_ref[...]
    b_ref[...] = jnp.dot(h, w1j_ref[...], preferred_element_type=jnp.float32)


def _node1(x, s, cnt, pw_t, pb, w1i_t, sb1, w1j_t):
    n, d = x.shape
    hdim = pw_t.shape[1]
    return pl.pallas_call(
        _node1_body,
        out_shape=(jax.ShapeDtypeStruct((n, hdim), jnp.float32),
                   jax.ShapeDtypeStruct((n, hdim), jnp.float32),
                   jax.ShapeDtypeStruct((n, hdim), jnp.float32)),
    )(x, s[0], s[1], cnt, pw_t, pb.reshape(1, -1), w1i_t, sb1.reshape(1, -1),
      w1j_t)


def _bn_relu(h, g, b):
    m = jnp.mean(h, axis=0, keepdims=True)
    v = jnp.mean((h - m) ** 2, axis=0, keepdims=True)
    return jnp.maximum((h - m) * jax.lax.rsqrt(v + 1e-5) * g + b, 0.0)


def _node2_body(h_ref, sa_ref, sb_ref, c_ref, g_ref, bb_ref, w1i_ref, sb1_ref,
                w1j_ref, h_out_ref, a_ref, b_ref):
    s, c = _part_sum(sa_ref, sb_ref, c_ref, h_ref.shape[0])
    h = h_ref[...] * (1.0 + s / c)
    hn = _bn_relu(h, g_ref[...], bb_ref[...])
    h_out_ref[...] = hn
    a_ref[...] = jnp.dot(hn, w1i_ref[...], preferred_element_type=jnp.float32) + sb1_ref[...]
    b_ref[...] = jnp.dot(hn, w1j_ref[...], preferred_element_type=jnp.float32)


def _node2(h, s, cnt, bn_g, bn_b, w1i_t, sb1, w1j_t):
    n, hdim = h.shape
    return pl.pallas_call(
        _node2_body,
        out_shape=(jax.ShapeDtypeStruct((n, hdim), jnp.float32),
                   jax.ShapeDtypeStruct((n, hdim), jnp.float32),
                   jax.ShapeDtypeStruct((n, hdim), jnp.float32)),
    )(h, s[0], s[1], cnt, bn_g.reshape(1, -1), bn_b.reshape(1, -1),
      w1i_t, sb1.reshape(1, -1), w1j_t)


def _node3_body(h_ref, sa_ref, sb_ref, c_ref, g_ref, bb_ref,
                w1_ref, b1_ref, g1_ref, bb1_ref,
                w2_ref, b2_ref, g2_ref, bb2_ref,
                w3_ref, b3_ref, out_ref):
    s, c = _part_sum(sa_ref, sb_ref, c_ref, h_ref.shape[0])
    h = h_ref[...] * (1.0 + s / c)
    hn = _bn_relu(h, g_ref[...], bb_ref[...])
    z = jnp.dot(hn, w1_ref[...], preferred_element_type=jnp.float32) + b1_ref[...]
    z = _bn_relu(z, g1_ref[...], bb1_ref[...])
    z = jnp.dot(z, w2_ref[...], preferred_element_type=jnp.float32) + b2_ref[...]
    z = _bn_relu(z, g2_ref[...], bb2_ref[...])
    out_ref[...] = jnp.dot(z, w3_ref[...], preferred_element_type=jnp.float32) + b3_ref[...]


def _node3(h, s, cnt, bn_g, bn_b, clf):
    n = h.shape[0]
    return pl.pallas_call(
        _node3_body,
        out_shape=jax.ShapeDtypeStruct((n, 1), jnp.float32),
    )(h, s[0], s[1], cnt, bn_g.reshape(1, -1), bn_b.reshape(1, -1),
      clf['W1'].T, clf['b1'].reshape(1, -1), clf['bn1_g'].reshape(1, -1), clf['bn1_b'].reshape(1, -1),
      clf['W2'].T, clf['b2'].reshape(1, -1), clf['bn2_g'].reshape(1, -1), clf['bn2_b'].reshape(1, -1),
      clf['W3'].T, clf['b3'].reshape(1, -1))


def _edge1_body(hid, g_ref, d_ref, freq_ref, ph_ref, wrel_ref, w2_ref, b2_ref,
                q_ref):
    g = g_ref[...]
    if hid < g.shape[1]:
        lane = jax.lax.broadcasted_iota(jnp.int32, g.shape, 1)
        g = jnp.where(lane < hid, g, 0.0)
    rel = jnp.cos(d_ref[...] * freq_ref[...] + ph_ref[...])
    hmid = jnp.maximum(
        g + jnp.dot(rel, wrel_ref[...], preferred_element_type=jnp.float32), 0.0)
    p = jnp.tanh(jnp.dot(hmid.astype(jnp.bfloat16),
                         w2_ref[...].astype(jnp.bfloat16),
                         preferred_element_type=jnp.float32) + b2_ref[...])
    q_ref[...] = 2.0 * p - 1.0


def _edge1(gsum, dts2d, freq, phase, wrel_t, w2_t, b2, hid):
    e, hdim = gsum.shape
    dout = w2_t.shape[1]
    t = freq.shape[0]
    grid = e // _BE
    return pl.pallas_call(
        functools.partial(_edge1_body, hid),
        grid=(grid,),
        in_specs=[
            pl.BlockSpec((_BE, hdim), lambda i: (i, 0)),
            pl.BlockSpec((_BE, 1), lambda i: (i, 0)),
            pl.BlockSpec((1, t), lambda i: (0, 0)),
            pl.BlockSpec((1, t), lambda i: (0, 0)),
            pl.BlockSpec((t, hdim), lambda i: (0, 0)),
            pl.BlockSpec((hdim, dout), lambda i: (0, 0)),
            pl.BlockSpec((1, dout), lambda i: (0, 0)),
        ],
        out_specs=pl.BlockSpec((_BE, dout), lambda i: (i, 0)),
        out_shape=jax.ShapeDtypeStruct((e, dout), jnp.float32),
    )(gsum, dts2d, freq.reshape(1, -1), phase.reshape(1, -1), wrel_t, w2_t,
      b2.reshape(1, -1))


def _edge2_body(g_ref, w2_ref, b2_ref, q_ref):
    hmid = jnp.maximum(g_ref[...], 0.0)
    p = jnp.tanh(jnp.dot(hmid.astype(jnp.bfloat16),
                         w2_ref[...].astype(jnp.bfloat16),
                         preferred_element_type=jnp.float32) + b2_ref[...])
    q_ref[...] = 2.0 * p - 1.0


def _edge2(gsum, w2_t, b2):
    e, hdim = gsum.shape
    dout = w2_t.shape[1]
    grid = e // _BE
    return pl.pallas_call(
        _edge2_body,
        grid=(grid,),
        in_specs=[
            pl.BlockSpec((_BE, hdim), lambda i: (i, 0)),
            pl.BlockSpec((hdim, dout), lambda i: (0, 0)),
            pl.BlockSpec((1, dout), lambda i: (0, 0)),
        ],
        out_specs=pl.BlockSpec((_BE, dout), lambda i: (i, 0)),
        out_shape=jax.ShapeDtypeStruct((e, dout), jnp.float32),
    )(gsum, w2_t, b2.reshape(1, -1))


_NC = 2
_NS = 16
_NW = _NC * _NS
_IB = 100


def _sc_mesh():
    return plsc.VectorSubcoreMesh(core_axis_name="c", subcore_axis_name="s",
                                  num_cores=_NC, num_subcores=_NS)


_SC_PARAMS = pltpu.CompilerParams(use_tc_tiling_on_sc=False)


def _batches(ew, ch):
    out, o = [], 0
    while o < ew:
        b = min(ch, ew - o)
        out.append((o, b))
        o += b
    return out


_SB = 624


def _zero_stripe(zbuf, acc, sid, n, dh, zr):
    def zrow(r, _):
        for t in range(dh // 16):
            zbuf[r, pl.ds(t * 16, 16)] = jnp.zeros((16,), jnp.float32)
        return 0

    lax.fori_loop(0, zr, zrow, 0)
    start = sid * _SB

    def zcopy(t, _):
        pltpu.sync_copy(zbuf, acc.at[pl.ds(start + t * zr, zr)])
        return 0

    lax.fori_loop(0, _SB // zr, zcopy, 0)
    tail = n - _NS * _SB

    @pl.when(sid == _NS - 1)
    def _():
        pltpu.sync_copy(zbuf.at[pl.ds(0, tail)], acc.at[pl.ds(_NS * _SB, tail)])


def _copy_out(acc, out_hbm, cid, sid, n):
    start = sid * _SB
    pltpu.sync_copy(acc.at[pl.ds(start, _SB)],
                    out_hbm.at[pl.ds(cid * n + start, _SB)])
    tail = n - _NS * _SB

    @pl.when(sid == _NS - 1)
    def _():
        pltpu.sync_copy(acc.at[pl.ds(_NS * _SB, tail)],
                        out_hbm.at[pl.ds(cid * n + _NS * _SB, tail)])


def _gather_sum(a_tbl, b_tbl, dst1, src1, out_dh=None):
    n, dh = a_tbl.shape
    out_dh = out_dh or dh
    e = dst1.shape[0]
    ew = e // _NW
    ch = 400

    @functools.partial(
        pl.kernel, mesh=_sc_mesh(),
        out_type=jax.ShapeDtypeStruct((e, out_dh), jnp.float32),
        compiler_params=_SC_PARAMS,
        scratch_types=[
            pltpu.VMEM((ew,), jnp.int32),
            pltpu.VMEM((ew,), jnp.int32),
            pltpu.VMEM((ch, dh), jnp.float32),
            pltpu.VMEM((ch, dh), jnp.float32),
            pltpu.SemaphoreType.DMA,
            pltpu.SemaphoreType.DMA,
        ],
    )
    def k(a_hbm, b_hbm, dst_hbm, src_hbm, out_hbm, idxa, idxb, bufa, bufb,
          sema, semb):
        wid = lax.axis_index("s") * _NC + lax.axis_index("c")
        pltpu.sync_copy(dst_hbm.at[pl.ds(wid * ew, ew)], idxa)
        pltpu.sync_copy(src_hbm.at[pl.ds(wid * ew, ew)], idxb)

        for o, ne in _batches(ew, ch):
            esl = pl.ds(0, ne)
            cpa = pltpu.async_copy(a_hbm.at[idxa.at[pl.ds(o, ne)]],
                                   bufa.at[esl], sema)
            cpb = pltpu.async_copy(b_hbm.at[idxb.at[pl.ds(o, ne)]],
                                   bufb.at[esl], semb)
            cpa.wait()
            cpb.wait()

            def row(r, _):
                for t in range(dh // 16):
                    sl = pl.ds(t * 16, 16)
                    bufa[r, sl] = bufa[r, sl] + bufb[r, sl]
                return 0

            lax.fori_loop(0, ne, row, 0)
            rsl = pl.ds(wid * ew + o, ne)
            if out_dh == dh:
                pltpu.sync_copy(bufa.at[esl], out_hbm.at[rsl])
            else:
                pltpu.sync_copy(bufa.at[esl], out_hbm.at[rsl, pl.ds(0, dh)])

    return k(a_tbl, b_tbl, dst1, src1)


def _sc_scatter(q, dst1, n):
    e, dh = q.shape
    ew = e // _NW
    zr = 16
    ch = 1600
    cs = 4
    cw = dh // cs

    @functools.partial(
        pl.kernel, mesh=_sc_mesh(),
        out_type=jax.ShapeDtypeStruct((_NC * n, dh), jnp.float32),
        compiler_params=_SC_PARAMS,
        scratch_types=[
            pltpu.VMEM((ew,), jnp.int32),
            pltpu.VMEM((ch, cw), jnp.float32),
            pltpu.VMEM((zr, cw), jnp.float32),
            pltpu.VMEM_SHARED((n, cw), jnp.float32),
        ],
    )
    def k(q_hbm, dst_hbm, out_hbm, idx, qbuf, zbuf, acc):
        cid = lax.axis_index("c")
        sid = lax.axis_index("s")
        wid = sid * _NC + cid
        pltpu.sync_copy(dst_hbm.at[pl.ds(wid * ew, ew)], idx)
        for p in range(cs):
            csl = pl.ds(p * cw, cw)
            _zero_stripe(zbuf, acc, sid, n, cw, zr)
            plsc.subcore_barrier()

            for o, ne in _batches(ew, ch):
                pltpu.sync_copy(q_hbm.at[pl.ds(wid * ew + o, ne), csl],
                                qbuf.at[pl.ds(0, ne)])
                pltpu.sync_copy(qbuf.at[pl.ds(0, ne)],
                                acc.at[idx.at[pl.ds(o, ne)]], add=True)
            plsc.subcore_barrier()
            start = sid * _SB
            pltpu.sync_copy(acc.at[pl.ds(start, _SB)],
                            out_hbm.at[pl.ds(cid * n + start, _SB), csl])
            tail = n - _NS * _SB

            @pl.when(sid == _NS - 1)
            def _():
                pltpu.sync_copy(acc.at[pl.ds(_NS * _SB, tail)],
                                out_hbm.at[pl.ds(cid * n + _NS * _SB, tail),
                                           csl])
            plsc.subcore_barrier()

    return k(q, dst1)


def _edge_counts(dst1, n):
    e = dst1.shape[0]
    ew = e // _NW
    zr = 16
    dh = 16
    ch = 1600

    @functools.partial(
        pl.kernel, mesh=_sc_mesh(),
        out_type=jax.ShapeDtypeStruct((_NC * n, dh), jnp.float32),
        compiler_params=_SC_PARAMS,
        scratch_types=[
            pltpu.VMEM((ew,), jnp.int32),
            pltpu.VMEM((ch, dh), jnp.float32),
            pltpu.VMEM((zr, dh), jnp.float32),
            pltpu.VMEM_SHARED((n, dh), jnp.float32),
        ],
    )
    def k(dst_hbm, out_hbm, idx, ones, zbuf, acc):
        cid = lax.axis_index("c")
        sid = lax.axis_index("s")
        wid = sid * _NC + cid
        pltpu.sync_copy(dst_hbm.at[pl.ds(wid * ew, ew)], idx)

        def fill(r, _):
            ones[r, pl.ds(0, 16)] = jnp.ones((16,), jnp.float32)
            return 0

        lax.fori_loop(0, ch, fill, 0)
        _zero_stripe(zbuf, acc, sid, n, dh, zr)
        plsc.subcore_barrier()

        for o, ne in _batches(ew, ch):
            pltpu.sync_copy(ones.at[pl.ds(0, ne)],
                            acc.at[idx.at[pl.ds(o, ne)]], add=True)
        plsc.subcore_barrier()
        _copy_out(acc, out_hbm, cid, sid, n)

    return k(dst1)


def kernel(x, dts, params, edge_index):
    src = edge_index[0]
    dst = edge_index[1]
    n, d = x.shape
    e = dst.shape[0]
    t = params['basis_freq'].shape[0]

    w1 = params['tmp_W1']
    hid = w1.shape[0]
    hpad = -hid % 128
    w1d_t = w1[:, :d].T
    w1s_t = w1[:, d:2 * d].T
    w1rel_t = jnp.pad(w1[:, 2 * d:].T, ((0, 0), (0, hpad)))
    tmp_b1 = params['tmp_b1']
    tmp_w2_t = jnp.pad(params['tmp_W2'].T, ((0, hpad), (0, 0)))

    eh = e // 2
    dts2d = dts.reshape(-1, 1)
    hv = []
    for i in range(2):
        sl = slice(i * eh, (i + 1) * eh)
        hv.append((dst[sl], src[sl], dts2d[sl]))

    cnt = _edge_counts(dst, n)

    a1, b1t = _node_pre(x, w1d_t, w1s_t, tmp_b1)
    s1 = []
    for d3, s3_, dt in hv:
        g = _gather_sum(a1, b1t, d3, s3_, out_dh=hid + hpad)
        q = _edge1(g, dt, params['basis_freq'], params['phase'],
                   w1rel_t, tmp_w2_t, params['tmp_b2'], hid)
        s1.append(_sc_scatter(q, d3, n))

    smp0, smp1 = params['smp']
    h, a2, b2t = _node1(x, s1, cnt, params['proj_W'].T, params['proj_b'],
                        smp0['W1'][:, :d].T, smp0['b1'], smp0['W1'][:, d:].T)

    s2 = []
    for d3, s3_, _ in hv:
        g = _gather_sum(a2, b2t, d3, s3_)
        q = _edge2(g, smp0['W2'].T, smp0['b2'])
        s2.append(_sc_scatter(q, d3, n))
    h, a3, b3t = _node2(h, s2, cnt, smp0['bn_g'], smp0['bn_b'],
                        smp1['W1'][:, :d].T, smp1['b1'], smp1['W1'][:, d:].T)

    s3 = []
    for d3, s3_, _ in hv:
        g = _gather_sum(a3, b3t, d3, s3_)
        q = _edge2(g, smp1['W2'].T, smp1['b2'])
        s3.append(_sc_scatter(q, d3, n))

    return _node3(h, s3, cnt, smp1['bn_g'], smp1['bn_b'], params['clf'])
